# Initial kernel scaffold; baseline (speedup 1.0000x reference)
import jax
import jax.numpy as jnp
from jax.experimental import pallas as pl


def kernel(init_embed, pca_W, pca_b, rel, mu_W1, mu_b1, mu_W2, mu_b2, lv_W1, lv_b1, lv_W2, lv_b2, edge_index, edge_type, sub):
    raise NotImplementedError("write your pallas kernel here")


# speedup vs baseline: 324.6125x; 324.6125x over previous
import functools

import jax
import jax.numpy as jnp
from jax import lax
from jax.experimental import pallas as pl
from jax.experimental.pallas import tpu as pltpu
from jax.experimental.pallas import tpu_sc as plsc

_N = 50000
_E = 800000
_INIT_DIM = 128
_D = 32
_F = 2
_NREL = 100
_B = 1024
_H = 16
_FD = _F * _D

_NC, _NS, _L = 2, 16, 16
_NW = _NC * _NS

_BLK = 2048
_NBLK = 13
_CHUNK = _BLK * _NBLK
_PADE = _CHUNK * _NW
_BATCH = 128
_WROW = 80
_NPOS = _N + _L
_AGG_ROWS = _B + 16
_PACK_CAP = _CHUNK + _BATCH


def _xpre_body(x_ref, w_ref, b_ref, o_ref):
    acc = jnp.dot(x_ref[...], w_ref[...], preferred_element_type=jnp.float32)
    o_ref[...] = jnp.tanh(acc + b_ref[...])


def _compute_xpre(init_embed, pca_W, pca_b):
    blk = 1000
    grid = _N // blk
    return pl.pallas_call(
        _xpre_body,
        grid=(grid,),
        in_specs=[
            pl.BlockSpec((blk, _INIT_DIM), lambda i: (i, 0)),
            pl.BlockSpec((_INIT_DIM, _FD), lambda i: (0, 0)),
            pl.BlockSpec((1, _FD), lambda i: (0, 0)),
        ],
        out_specs=pl.BlockSpec((blk, _FD), lambda i: (i, 0)),
        out_shape=jax.ShapeDtypeStruct((_N, _FD), jnp.float32),
    )(init_embed, pca_W, pca_b.reshape(1, _FD))


def _mp_body(dst_hbm, src_hbm, et_hbm, xpre_hbm, relx2_hbm, sub_hbm, zeros_hbm,
             agg_out, slotmap_out, subx_out,
             pos_v, sub_v, dst_v, pack_v, gid_v, slot_v, srcv_v, etv_v,
             xrows_v, relrows_v, msg_v, subxc_v, slotc_v, agg_sh):
    cid = lax.axis_index("c")
    sid = lax.axis_index("s")
    wid = sid * _NC + cid
    base = wid * _CHUNK
    iota = lax.iota(jnp.int32, _L)

    pltpu.sync_copy(sub_hbm, sub_v)

    @pl.when(sid == 0)
    def _zero():
        pltpu.sync_copy(zeros_hbm, agg_sh)

    neg1 = jnp.full((_L,), -1, jnp.int32)

    def _memset(i, c):
        pos_v[pl.ds(i * _L, _L)] = neg1
        return c
    lax.fori_loop(0, _NPOS // _L, _memset, jnp.int32(0))

    def _scatter_pos(i, c):
        nodes = sub_v[pl.ds(i * _L, _L)]
        plsc.store_scatter(pos_v, [nodes], i * _L + iota)
        return c
    lax.fori_loop(0, _B // _L, _scatter_pos, jnp.int32(0))

    plsc.subcore_barrier()

    def _block(bi, cnt):
        pltpu.sync_copy(dst_hbm.at[pl.ds(base + bi * _BLK, _BLK)], dst_v)

        def _vec(i, cnt):
            d = dst_v[pl.ds(i * _L, _L)]
            p = plsc.load_gather(pos_v, [d])
            m = p >= 0
            gid = base + bi * _BLK + i * _L + iota
            packv = gid * 2048 + p
            plsc.store_compressed(pack_v.at[pl.ds(cnt, _L)], packv, mask=m)
            pcv = plsc.all_reduce_population_count(m)
            return cnt + pcv[0]
        return lax.fori_loop(0, _BLK // _L, _vec, cnt)
    cnt = lax.fori_loop(0, _NBLK, _block, jnp.int32(0))

    dummy = jnp.full((_L,), 1024, jnp.int32)

    def _pad(i, c):
        pack_v[pl.ds(cnt + i * _L, _L)] = dummy
        return c
    lax.fori_loop(0, _BATCH // _L, _pad, jnp.int32(0))
    nb = lax.div(cnt + jnp.int32(_BATCH - 1), jnp.int32(_BATCH))

    deg_vec = jnp.where(iota == 0, 1.0, 0.0).astype(jnp.float32)

    def _deg_init(e, c):
        msg_v[e, pl.ds(_FD, _L)] = deg_vec
        return c
    lax.fori_loop(0, _BATCH, _deg_init, jnp.int32(0))

    def _batch(bi, c):
        off = bi * _BATCH

        def _unpack(i, c2):
            pk = pack_v[pl.ds(off + i * _L, _L)]
            gid_v[pl.ds(i * _L, _L)] = lax.shift_right_logical(pk, 11)
            slot_v[pl.ds(i * _L, _L)] = lax.bitwise_and(pk, jnp.int32(2047))
            return c2
        lax.fori_loop(0, _BATCH // _L, _unpack, jnp.int32(0))

        pltpu.sync_copy(src_hbm.at[gid_v], srcv_v)
        pltpu.sync_copy(et_hbm.at[gid_v], etv_v)
        pltpu.sync_copy(xpre_hbm.at[srcv_v], xrows_v)
        pltpu.sync_copy(relx2_hbm.at[etv_v], relrows_v)

        def _mul(e, c2):
            for j in range(_FD // _L):
                a = xrows_v[e, pl.ds(j * _L, _L)]
                r = relrows_v[e, pl.ds(j * _L, _L)]
                msg_v[e, pl.ds(j * _L, _L)] = a * r
            return c2
        lax.fori_loop(0, _BATCH, _mul, jnp.int32(0))

        pltpu.sync_copy(msg_v, agg_sh.at[slot_v], add=True)
        return c
    lax.fori_loop(0, nb, _batch, jnp.int32(0))

    plsc.subcore_barrier()

    @pl.when(sid == 0)
    def _flush():
        pltpu.sync_copy(agg_sh, agg_out.at[cid])

    rbase = wid * 32
    pltpu.sync_copy(xpre_hbm.at[sub_v.at[pl.ds(rbase, 32)]], subxc_v)
    pltpu.sync_copy(subxc_v, subx_out.at[pl.ds(rbase, 32)])
    s0 = sub_v[pl.ds(rbase, _L)]
    s1 = sub_v[pl.ds(rbase + _L, _L)]
    slotc_v[pl.ds(0, _L)] = plsc.load_gather(pos_v, [s0])
    slotc_v[pl.ds(_L, _L)] = plsc.load_gather(pos_v, [s1])
    pltpu.sync_copy(slotc_v, slotmap_out.at[pl.ds(rbase, 32)])


def _run_mp(dst_pad, src, et, xpre, relx2, sub, zeros):
    mesh = plsc.VectorSubcoreMesh(
        core_axis_name="c", subcore_axis_name="s",
        num_cores=_NC, num_subcores=_NS)
    f = functools.partial(
        pl.kernel,
        out_type=[
            jax.ShapeDtypeStruct((_NC, _AGG_ROWS, _WROW), jnp.float32),
            jax.ShapeDtypeStruct((_B,), jnp.int32),
            jax.ShapeDtypeStruct((_B, _FD), jnp.float32),
        ],
        mesh=mesh,
        compiler_params=pltpu.CompilerParams(
            needs_layout_passes=False, use_tc_tiling_on_sc=False),
        scratch_types=[
            pltpu.VMEM((_NPOS,), jnp.int32),
            pltpu.VMEM((_B,), jnp.int32),
            pltpu.VMEM((_BLK,), jnp.int32),
            pltpu.VMEM((_PACK_CAP,), jnp.int32),
            pltpu.VMEM((_BATCH,), jnp.int32),
            pltpu.VMEM((_BATCH,), jnp.int32),
            pltpu.VMEM((_BATCH,), jnp.int32),
            pltpu.VMEM((_BATCH,), jnp.int32),
            pltpu.VMEM((_BATCH, _FD), jnp.float32),
            pltpu.VMEM((_BATCH, _FD), jnp.float32),
            pltpu.VMEM((_BATCH, _WROW), jnp.float32),
            pltpu.VMEM((32, _FD), jnp.float32),
            pltpu.VMEM((32,), jnp.int32),
            pltpu.VMEM_SHARED((_AGG_ROWS, _WROW), jnp.float32),
        ],
    )(_mp_body)
    return f(dst_pad, src, et, xpre, relx2, sub, zeros)


def _club_body(agg2_ref, slot_ref, perm_ref, subx_ref,
               muW1, mub1, muW2, mub2, lvW1, lvb1, lvW2, lvb2, o_ref):
    agg = agg2_ref[0] + agg2_ref[1]
    oh = (slot_ref[...] == lax.broadcasted_iota(
        jnp.int32, (_B, _AGG_ROWS), 1)).astype(jnp.float32)
    sel = jnp.dot(oh, agg, preferred_element_type=jnp.float32)
    msg = sel[:, :_FD]
    deg = jnp.maximum(sel[:, _FD:_FD + 1], 1.0)
    xsub = jnp.tanh(subx_ref[...] + msg / deg)
    xs = xsub[:, :_D]
    ys = xsub[:, _D:_FD]
    hmu = jnp.maximum(jnp.dot(xs, muW1[...], preferred_element_type=jnp.float32)
                      + mub1[...], 0.0)
    mu = jnp.dot(hmu, muW2[...], preferred_element_type=jnp.float32) + mub2[...]
    hlv = jnp.maximum(jnp.dot(xs, lvW1[...], preferred_element_type=jnp.float32)
                      + lvb1[...], 0.0)
    logvar = jnp.tanh(jnp.dot(hlv, lvW2[...], preferred_element_type=jnp.float32)
                      + lvb2[...])
    inv = jnp.exp(-logvar)
    ohp = (perm_ref[...] == lax.broadcasted_iota(
        jnp.int32, (_B, _B), 1)).astype(jnp.float32)
    ysp = jnp.dot(ohp, ys, preferred_element_type=jnp.float32)
    pos_t = ((mu - ys) ** 2) * inv
    neg_t = ((mu - ysp) ** 2) * inv
    val = (jnp.sum(neg_t) - jnp.sum(pos_t)) / (2.0 * _B)
    o_ref[...] = val.reshape(1, 1)


def _run_club(agg2, slotmap, perm, subx,
              mu_W1, mu_b1, mu_W2, mu_b2, lv_W1, lv_b1, lv_W2, lv_b2):
    return pl.pallas_call(
        _club_body,
        out_shape=jax.ShapeDtypeStruct((1, 1), jnp.float32),
    )(agg2, slotmap.reshape(_B, 1), perm.reshape(_B, 1), subx,
      mu_W1, mu_b1.reshape(1, _H), mu_W2, mu_b2.reshape(1, _D),
      lv_W1, lv_b1.reshape(1, _H), lv_W2, lv_b2.reshape(1, _D))


def kernel(init_embed, pca_W, pca_b, rel, mu_W1, mu_b1, mu_W2, mu_b2,
           lv_W1, lv_b1, lv_W2, lv_b2, edge_index, edge_type, sub):
    xpre = _compute_xpre(init_embed, pca_W, pca_b)

    src = edge_index[0]
    dst = edge_index[1]
    dst_pad = jnp.concatenate(
        [dst, jnp.full((_PADE - _E,), _N, dtype=jnp.int32)])
    relx2 = jnp.tile(rel, (1, _F))
    zeros = jnp.zeros((_AGG_ROWS, _WROW), jnp.float32)

    agg2, slotmap, subx = _run_mp(dst_pad, src, edge_type, xpre, relx2, sub,
                                  zeros)

    perm = jax.random.permutation(jax.random.key(1), _B).astype(jnp.int32)
    out = _run_club(agg2, slotmap, perm, subx,
                    mu_W1, mu_b1, mu_W2, mu_b2, lv_W1, lv_b1, lv_W2, lv_b2)
    return out.reshape(())


# speedup vs baseline: 367.8046x; 1.1331x over previous
import functools

import jax
import jax.numpy as jnp
from jax import lax
from jax.experimental import pallas as pl
from jax.experimental.pallas import tpu as pltpu
from jax.experimental.pallas import tpu_sc as plsc

_N = 50000
_E = 800000
_INIT_DIM = 128
_D = 32
_F = 2
_NREL = 100
_B = 1024
_H = 16
_FD = _F * _D

_NC, _NS, _L = 2, 16, 16
_NW = _NC * _NS

_BLK = 2048
_CHUNK = _E // _NW
_NFULL = _CHUNK // _BLK
_TAIL = _CHUNK - _NFULL * _BLK
_BATCH = 128
_WROW = 80
_NPOS = _N + _L
_AGG_ROWS = _B + 16
_CFILT = _BLK + _BATCH + _L
_TRASH = 1024 * 256


def _xpre_body(x_ref, w_ref, b_ref, o_ref):
    acc = jnp.dot(x_ref[...], w_ref[...], preferred_element_type=jnp.float32, precision=lax.Precision.HIGHEST)
    o_ref[...] = jnp.tanh(acc + b_ref[...])


def _compute_xpre(init_embed, pca_W, pca_b):
    blk = 2000
    grid = _N // blk
    return pl.pallas_call(
        _xpre_body,
        grid=(grid,),
        in_specs=[
            pl.BlockSpec((blk, _INIT_DIM), lambda i: (i, 0)),
            pl.BlockSpec((_INIT_DIM, _FD), lambda i: (0, 0)),
            pl.BlockSpec((1, _FD), lambda i: (0, 0)),
        ],
        out_specs=pl.BlockSpec((blk, _FD), lambda i: (i, 0)),
        out_shape=jax.ShapeDtypeStruct((_N, _FD), jnp.float32),
    )(init_embed, pca_W, pca_b.reshape(1, _FD))


def _mp_body(ei_hbm, et_hbm, xpre_hbm, relx2_hbm, sub_hbm, negones_hbm,
             zeros_hbm,
             agg_out, slotmap_out, subx_out,
             pos_v, sub_v, dstb_v, srcb_v, etb_v, srcf_v, psef_v,
             slot_v, etv_v, xrows_v, relrows_v, msg_v, subxc_v, slotc_v,
             agg_sh):
    cid = lax.axis_index("c")
    sid = lax.axis_index("s")
    wid = sid * _NC + cid
    base = wid * _CHUNK
    iota = lax.iota(jnp.int32, _L)

    pltpu.sync_copy(sub_hbm, sub_v)
    pltpu.sync_copy(negones_hbm, pos_v)

    @pl.when(sid == 0)
    def _zero():
        pltpu.sync_copy(zeros_hbm, agg_sh)

    def _scatter_pos(i, c):
        nodes = sub_v[pl.ds(i * _L, _L)]
        plsc.store_scatter(pos_v, [nodes], i * _L + iota)
        return c
    lax.fori_loop(0, _B // _L, _scatter_pos, jnp.int32(0))

    deg_vec = jnp.where(iota == 0, 1.0, 0.0).astype(jnp.float32)

    def _deg_init(e, c):
        msg_v[e, pl.ds(_FD, _L)] = deg_vec
        return c
    lax.fori_loop(0, _BATCH, _deg_init, jnp.int32(0))

    plsc.subcore_barrier()

    def _batch(bi, c):
        off = bi * _BATCH

        def _unp(i, c2):
            v = psef_v[pl.ds(off + i * _L, _L)]
            slot_v[pl.ds(i * _L, _L)] = lax.shift_right_logical(v, 8)
            etv_v[pl.ds(i * _L, _L)] = lax.bitwise_and(v, jnp.int32(255))
            return c2
        lax.fori_loop(0, _BATCH // _L, _unp, jnp.int32(0))

        pltpu.sync_copy(xpre_hbm.at[srcf_v.at[pl.ds(off, _BATCH)]], xrows_v)
        pltpu.sync_copy(relx2_hbm.at[etv_v], relrows_v)

        def _mul(e, c2):
            for j in range(_FD // _L):
                a = xrows_v[e, pl.ds(j * _L, _L)]
                r = relrows_v[e, pl.ds(j * _L, _L)]
                msg_v[e, pl.ds(j * _L, _L)] = a * r
            return c2
        lax.fori_loop(0, _BATCH, _mul, jnp.int32(0), unroll=2)

        pltpu.sync_copy(msg_v, agg_sh.at[slot_v], add=True)
        return c

    def _do_block(off_words, thr, cnt):
        pltpu.sync_copy(ei_hbm.at[1, pl.ds(off_words, _BLK)], dstb_v)
        pltpu.sync_copy(ei_hbm.at[0, pl.ds(off_words, _BLK)], srcb_v)
        pltpu.sync_copy(et_hbm.at[pl.ds(off_words, _BLK)], etb_v)

        def _vec(i, cnt):
            d = dstb_v[pl.ds(i * _L, _L)]
            p = plsc.load_gather(pos_v, [d])
            lane = i * _L + iota
            m = jnp.logical_and(p >= 0, lane >= thr)
            s = srcb_v[pl.ds(i * _L, _L)]
            t = etb_v[pl.ds(i * _L, _L)]
            pse = p * 256 + t
            plsc.store_compressed(srcf_v.at[pl.ds(cnt, _L)], s, mask=m)
            plsc.store_compressed(psef_v.at[pl.ds(cnt, _L)], pse, mask=m)
            return cnt + plsc.all_reduce_population_count(m)[0]
        cnt = lax.fori_loop(0, _BLK // _L, _vec, cnt, unroll=4)

        nb = lax.div(cnt, jnp.int32(_BATCH))
        lax.fori_loop(0, nb, _batch, jnp.int32(0))
        rem = cnt - nb * _BATCH

        def _mv(i, c):
            g = i * _L + iota
            v1 = srcf_v[pl.ds(nb * _BATCH + i * _L, _L)]
            v2 = psef_v[pl.ds(nb * _BATCH + i * _L, _L)]
            srcf_v[pl.ds(i * _L, _L)] = jnp.where(g < rem, v1, 0)
            psef_v[pl.ds(i * _L, _L)] = jnp.where(g < rem, v2,
                                                  jnp.int32(_TRASH))
            return c
        lax.fori_loop(0, _BATCH // _L, _mv, jnp.int32(0))
        return rem

    def _full_block(bi, cnt):
        return _do_block(base + bi * _BLK, jnp.int32(0), cnt)
    cnt = lax.fori_loop(0, _NFULL, _full_block, jnp.int32(0))
    cnt = _do_block(base + _CHUNK - _BLK, jnp.int32(_BLK - _TAIL), cnt)

    lax.fori_loop(0, lax.div(cnt + jnp.int32(_BATCH - 1), jnp.int32(_BATCH)),
                  _batch, jnp.int32(0))

    plsc.subcore_barrier()

    @pl.when(sid == 0)
    def _flush():
        pltpu.sync_copy(agg_sh, agg_out.at[cid])

    rbase = wid * 32
    pltpu.sync_copy(xpre_hbm.at[sub_v.at[pl.ds(rbase, 32)]], subxc_v)
    pltpu.sync_copy(subxc_v, subx_out.at[pl.ds(rbase, 32)])
    s0 = sub_v[pl.ds(rbase, _L)]
    s1 = sub_v[pl.ds(rbase + _L, _L)]
    slotc_v[pl.ds(0, _L)] = plsc.load_gather(pos_v, [s0])
    slotc_v[pl.ds(_L, _L)] = plsc.load_gather(pos_v, [s1])
    pltpu.sync_copy(slotc_v, slotmap_out.at[pl.ds(rbase, 32)])


def _run_mp(edge_index, edge_type, xpre, relx2, sub, negones, zeros):
    mesh = plsc.VectorSubcoreMesh(
        core_axis_name="c", subcore_axis_name="s",
        num_cores=_NC, num_subcores=_NS)
    f = functools.partial(
        pl.kernel,
        out_type=[
            jax.ShapeDtypeStruct((_NC, _AGG_ROWS, _WROW), jnp.float32),
            jax.ShapeDtypeStruct((_B,), jnp.int32),
            jax.ShapeDtypeStruct((_B, _FD), jnp.float32),
        ],
        mesh=mesh,
        compiler_params=pltpu.CompilerParams(
            needs_layout_passes=False, use_tc_tiling_on_sc=False),
        scratch_types=[
            pltpu.VMEM((_NPOS,), jnp.int32),
            pltpu.VMEM((_B,), jnp.int32),
            pltpu.VMEM((_BLK,), jnp.int32),
            pltpu.VMEM((_BLK,), jnp.int32),
            pltpu.VMEM((_BLK,), jnp.int32),
            pltpu.VMEM((_CFILT,), jnp.int32),
            pltpu.VMEM((_CFILT,), jnp.int32),
            pltpu.VMEM((_BATCH,), jnp.int32),
            pltpu.VMEM((_BATCH,), jnp.int32),
            pltpu.VMEM((_BATCH, _FD), jnp.float32),
            pltpu.VMEM((_BATCH, _FD), jnp.float32),
            pltpu.VMEM((_BATCH, _WROW), jnp.float32),
            pltpu.VMEM((32, _FD), jnp.float32),
            pltpu.VMEM((32,), jnp.int32),
            pltpu.VMEM_SHARED((_AGG_ROWS, _WROW), jnp.float32),
        ],
    )(_mp_body)
    return f(edge_index, edge_type, xpre, relx2, sub, negones, zeros)


def _club_body(agg2_ref, slot_ref, perm_ref, subx_ref,
               muW1, mub1, muW2, mub2, lvW1, lvb1, lvW2, lvb2, o_ref):
    agg = agg2_ref[0] + agg2_ref[1]
    oh = (slot_ref[...] == lax.broadcasted_iota(
        jnp.int32, (_B, _AGG_ROWS), 1)).astype(jnp.float32)
    sel = jnp.dot(oh, agg, preferred_element_type=jnp.float32, precision=lax.Precision.HIGHEST)
    msg = sel[:, :_FD]
    deg = jnp.maximum(sel[:, _FD:_FD + 1], 1.0)
    xsub = jnp.tanh(subx_ref[...] + msg / deg)
    xs = xsub[:, :_D]
    ys = xsub[:, _D:_FD]
    hmu = jnp.maximum(jnp.dot(xs, muW1[...], preferred_element_type=jnp.float32, precision=lax.Precision.HIGHEST)
                      + mub1[...], 0.0)
    mu = jnp.dot(hmu, muW2[...], preferred_element_type=jnp.float32, precision=lax.Precision.HIGHEST) + mub2[...]
    hlv = jnp.maximum(jnp.dot(xs, lvW1[...], preferred_element_type=jnp.float32, precision=lax.Precision.HIGHEST)
                      + lvb1[...], 0.0)
    logvar = jnp.tanh(jnp.dot(hlv, lvW2[...], preferred_element_type=jnp.float32, precision=lax.Precision.HIGHEST)
                      + lvb2[...])
    inv = jnp.exp(-logvar)
    ohp = (perm_ref[...] == lax.broadcasted_iota(
        jnp.int32, (_B, _B), 1)).astype(jnp.float32)
    ysp = jnp.dot(ohp, ys, preferred_element_type=jnp.float32, precision=lax.Precision.HIGHEST)
    pos_t = ((mu - ys) ** 2) * inv
    neg_t = ((mu - ysp) ** 2) * inv
    val = (jnp.sum(neg_t) - jnp.sum(pos_t)) / (2.0 * _B)
    o_ref[...] = val.reshape(1, 1)


def _run_club(agg2, slotmap, perm, subx,
              mu_W1, mu_b1, mu_W2, mu_b2, lv_W1, lv_b1, lv_W2, lv_b2):
    return pl.pallas_call(
        _club_body,
        out_shape=jax.ShapeDtypeStruct((1, 1), jnp.float32),
    )(agg2, slotmap.reshape(_B, 1), perm.reshape(_B, 1), subx,
      mu_W1, mu_b1.reshape(1, _H), mu_W2, mu_b2.reshape(1, _D),
      lv_W1, lv_b1.reshape(1, _H), lv_W2, lv_b2.reshape(1, _D))


def kernel(init_embed, pca_W, pca_b, rel, mu_W1, mu_b1, mu_W2, mu_b2,
           lv_W1, lv_b1, lv_W2, lv_b2, edge_index, edge_type, sub):
    xpre = _compute_xpre(init_embed, pca_W, pca_b)

    relx2 = jnp.tile(rel, (1, _F))
    negones = jnp.full((_NPOS,), -1, jnp.int32)
    zeros = jnp.zeros((_AGG_ROWS, _WROW), jnp.float32)

    agg2, slotmap, subx = _run_mp(edge_index, edge_type, xpre, relx2, sub,
                                  negones, zeros)

    perm = jax.random.permutation(jax.random.key(1), _B).astype(jnp.int32)
    out = _run_club(agg2, slotmap, perm, subx,
                    mu_W1, mu_b1, mu_W2, mu_b2, lv_W1, lv_b1, lv_W2, lv_b2)
    return out.reshape(())


# speedup vs baseline: 380.0087x; 1.0332x over previous
import functools

import jax
import jax.numpy as jnp
from jax import lax
from jax.experimental import pallas as pl
from jax.experimental.pallas import tpu as pltpu
from jax.experimental.pallas import tpu_sc as plsc

_N = 50000
_E = 800000
_INIT_DIM = 128
_D = 32
_F = 2
_NREL = 100
_B = 1024
_H = 16
_FD = _F * _D

_NC, _NS, _L = 2, 16, 16
_NW = _NC * _NS

_BLK = 2048
_CHUNK = _E // _NW
_NFULL = _CHUNK // _BLK
_TAIL = _CHUNK - _NFULL * _BLK
_BATCH = 128
_WROW = 80
_NPOS = _N + _L
_AGG_ROWS = _B + 16
_CFILT = _BLK + _BATCH + _L
_TRASH = 1024 * 256


def _xpre_body(x_ref, w_ref, b_ref, o_ref):
    acc = jnp.dot(x_ref[...], w_ref[...], preferred_element_type=jnp.float32, precision=lax.Precision.DEFAULT)
    o_ref[...] = jnp.tanh(acc + b_ref[...])


def _compute_xpre(init_embed, pca_W, pca_b):
    blk = 2000
    grid = _N // blk
    return pl.pallas_call(
        _xpre_body,
        grid=(grid,),
        in_specs=[
            pl.BlockSpec((blk, _INIT_DIM), lambda i: (i, 0)),
            pl.BlockSpec((_INIT_DIM, _FD), lambda i: (0, 0)),
            pl.BlockSpec((1, _FD), lambda i: (0, 0)),
        ],
        out_specs=pl.BlockSpec((blk, _FD), lambda i: (i, 0)),
        out_shape=jax.ShapeDtypeStruct((_N, _FD), jnp.float32),
    )(init_embed, pca_W, pca_b.reshape(1, _FD))


def _mp_body(ei_hbm, et_hbm, xpre_hbm, relx2_hbm, sub_hbm, negones_hbm,
             zeros_hbm,
             agg_out, slotmap_out, subx_out,
             pos_v, sub_v, dstb_v, srcb_v, etb_v, srcf_v, psef_v,
             slot_v, etv_v, xrows_v, relrows_v, msg_v, subxc_v, slotc_v,
             agg_sh):
    cid = lax.axis_index("c")
    sid = lax.axis_index("s")
    wid = sid * _NC + cid
    base = wid * _CHUNK
    iota = lax.iota(jnp.int32, _L)

    pltpu.sync_copy(sub_hbm, sub_v)
    pltpu.sync_copy(negones_hbm, pos_v)

    @pl.when(sid == 0)
    def _zero():
        pltpu.sync_copy(zeros_hbm, agg_sh)

    def _scatter_pos(i, c):
        nodes = sub_v[pl.ds(i * _L, _L)]
        plsc.store_scatter(pos_v, [nodes], i * _L + iota)
        return c
    lax.fori_loop(0, _B // _L, _scatter_pos, jnp.int32(0))

    deg_vec = jnp.where(iota == 0, 1.0, 0.0).astype(jnp.float32)

    def _deg_init(e, c):
        msg_v[e, pl.ds(_FD, _L)] = deg_vec
        return c
    lax.fori_loop(0, _BATCH, _deg_init, jnp.int32(0))

    plsc.subcore_barrier()

    def _batch(bi, c):
        off = bi * _BATCH

        def _unp(i, c2):
            v = psef_v[pl.ds(off + i * _L, _L)]
            slot_v[pl.ds(i * _L, _L)] = lax.shift_right_logical(v, 8)
            etv_v[pl.ds(i * _L, _L)] = lax.bitwise_and(v, jnp.int32(255))
            return c2
        lax.fori_loop(0, _BATCH // _L, _unp, jnp.int32(0))

        pltpu.sync_copy(xpre_hbm.at[srcf_v.at[pl.ds(off, _BATCH)]], xrows_v)
        pltpu.sync_copy(relx2_hbm.at[etv_v], relrows_v)

        def _mul(e, c2):
            for j in range(_FD // _L):
                a = xrows_v[e, pl.ds(j * _L, _L)]
                r = relrows_v[e, pl.ds(j * _L, _L)]
                msg_v[e, pl.ds(j * _L, _L)] = a * r
            return c2
        lax.fori_loop(0, _BATCH, _mul, jnp.int32(0), unroll=2)

        pltpu.sync_copy(msg_v, agg_sh.at[slot_v], add=True)
        return c

    def _do_block(off_words, thr, cnt):
        pltpu.sync_copy(ei_hbm.at[1, pl.ds(off_words, _BLK)], dstb_v)
        pltpu.sync_copy(ei_hbm.at[0, pl.ds(off_words, _BLK)], srcb_v)
        pltpu.sync_copy(et_hbm.at[pl.ds(off_words, _BLK)], etb_v)

        def _vec(i, cnt):
            d = dstb_v[pl.ds(i * _L, _L)]
            p = plsc.load_gather(pos_v, [d])
            lane = i * _L + iota
            m = jnp.logical_and(p >= 0, lane >= thr)
            s = srcb_v[pl.ds(i * _L, _L)]
            t = etb_v[pl.ds(i * _L, _L)]
            pse = p * 256 + t
            plsc.store_compressed(srcf_v.at[pl.ds(cnt, _L)], s, mask=m)
            plsc.store_compressed(psef_v.at[pl.ds(cnt, _L)], pse, mask=m)
            return cnt + plsc.all_reduce_population_count(m)[0]
        cnt = lax.fori_loop(0, _BLK // _L, _vec, cnt, unroll=4)

        nb = lax.div(cnt, jnp.int32(_BATCH))
        lax.fori_loop(0, nb, _batch, jnp.int32(0))
        rem = cnt - nb * _BATCH

        def _mv(i, c):
            g = i * _L + iota
            v1 = srcf_v[pl.ds(nb * _BATCH + i * _L, _L)]
            v2 = psef_v[pl.ds(nb * _BATCH + i * _L, _L)]
            srcf_v[pl.ds(i * _L, _L)] = jnp.where(g < rem, v1, 0)
            psef_v[pl.ds(i * _L, _L)] = jnp.where(g < rem, v2,
                                                  jnp.int32(_TRASH))
            return c
        lax.fori_loop(0, _BATCH // _L, _mv, jnp.int32(0))
        return rem

    def _full_block(bi, cnt):
        return _do_block(base + bi * _BLK, jnp.int32(0), cnt)
    cnt = lax.fori_loop(0, _NFULL, _full_block, jnp.int32(0))
    cnt = _do_block(base + _CHUNK - _BLK, jnp.int32(_BLK - _TAIL), cnt)

    lax.fori_loop(0, lax.div(cnt + jnp.int32(_BATCH - 1), jnp.int32(_BATCH)),
                  _batch, jnp.int32(0))

    plsc.subcore_barrier()

    @pl.when(sid == 0)
    def _flush():
        pltpu.sync_copy(agg_sh, agg_out.at[cid])

    rbase = wid * 32
    pltpu.sync_copy(xpre_hbm.at[sub_v.at[pl.ds(rbase, 32)]], subxc_v)
    pltpu.sync_copy(subxc_v, subx_out.at[pl.ds(rbase, 32)])
    s0 = sub_v[pl.ds(rbase, _L)]
    s1 = sub_v[pl.ds(rbase + _L, _L)]
    slotc_v[pl.ds(0, _L)] = plsc.load_gather(pos_v, [s0])
    slotc_v[pl.ds(_L, _L)] = plsc.load_gather(pos_v, [s1])
    pltpu.sync_copy(slotc_v, slotmap_out.at[pl.ds(rbase, 32)])


def _run_mp(edge_index, edge_type, xpre, relx2, sub, negones, zeros):
    mesh = plsc.VectorSubcoreMesh(
        core_axis_name="c", subcore_axis_name="s",
        num_cores=_NC, num_subcores=_NS)
    f = functools.partial(
        pl.kernel,
        out_type=[
            jax.ShapeDtypeStruct((_NC, _AGG_ROWS, _WROW), jnp.float32),
            jax.ShapeDtypeStruct((_B,), jnp.int32),
            jax.ShapeDtypeStruct((_B, _FD), jnp.float32),
        ],
        mesh=mesh,
        compiler_params=pltpu.CompilerParams(
            needs_layout_passes=False, use_tc_tiling_on_sc=False),
        scratch_types=[
            pltpu.VMEM((_NPOS,), jnp.int32),
            pltpu.VMEM((_B,), jnp.int32),
            pltpu.VMEM((_BLK,), jnp.int32),
            pltpu.VMEM((_BLK,), jnp.int32),
            pltpu.VMEM((_BLK,), jnp.int32),
            pltpu.VMEM((_CFILT,), jnp.int32),
            pltpu.VMEM((_CFILT,), jnp.int32),
            pltpu.VMEM((_BATCH,), jnp.int32),
            pltpu.VMEM((_BATCH,), jnp.int32),
            pltpu.VMEM((_BATCH, _FD), jnp.float32),
            pltpu.VMEM((_BATCH, _FD), jnp.float32),
            pltpu.VMEM((_BATCH, _WROW), jnp.float32),
            pltpu.VMEM((32, _FD), jnp.float32),
            pltpu.VMEM((32,), jnp.int32),
            pltpu.VMEM_SHARED((_AGG_ROWS, _WROW), jnp.float32),
        ],
    )(_mp_body)
    return f(edge_index, edge_type, xpre, relx2, sub, negones, zeros)


def _club_body(agg2_ref, slot_ref, perm_ref, subx_ref,
               muW1, mub1, muW2, mub2, lvW1, lvb1, lvW2, lvb2, o_ref):
    agg = agg2_ref[0] + agg2_ref[1]
    oh = (slot_ref[...] == lax.broadcasted_iota(
        jnp.int32, (_B, _AGG_ROWS), 1)).astype(jnp.float32)
    sel = jnp.dot(oh, agg, preferred_element_type=jnp.float32, precision=lax.Precision.HIGHEST)
    msg = sel[:, :_FD]
    deg = jnp.maximum(sel[:, _FD:_FD + 1], 1.0)
    xsub = jnp.tanh(subx_ref[...] + msg / deg)
    xs = xsub[:, :_D]
    ys = xsub[:, _D:_FD]
    hmu = jnp.maximum(jnp.dot(xs, muW1[...], preferred_element_type=jnp.float32, precision=lax.Precision.DEFAULT)
                      + mub1[...], 0.0)
    mu = jnp.dot(hmu, muW2[...], preferred_element_type=jnp.float32, precision=lax.Precision.DEFAULT) + mub2[...]
    hlv = jnp.maximum(jnp.dot(xs, lvW1[...], preferred_element_type=jnp.float32, precision=lax.Precision.DEFAULT)
                      + lvb1[...], 0.0)
    logvar = jnp.tanh(jnp.dot(hlv, lvW2[...], preferred_element_type=jnp.float32, precision=lax.Precision.DEFAULT)
                      + lvb2[...])
    inv = jnp.exp(-logvar)
    ohp = (perm_ref[...] == lax.broadcasted_iota(
        jnp.int32, (_B, _B), 1)).astype(jnp.float32)
    ysp = jnp.dot(ohp, ys, preferred_element_type=jnp.float32, precision=lax.Precision.HIGHEST)
    pos_t = ((mu - ys) ** 2) * inv
    neg_t = ((mu - ysp) ** 2) * inv
    val = (jnp.sum(neg_t) - jnp.sum(pos_t)) / (2.0 * _B)
    o_ref[...] = val.reshape(1, 1)


def _run_club(agg2, slotmap, perm, subx,
              mu_W1, mu_b1, mu_W2, mu_b2, lv_W1, lv_b1, lv_W2, lv_b2):
    return pl.pallas_call(
        _club_body,
        out_shape=jax.ShapeDtypeStruct((1, 1), jnp.float32),
    )(agg2, slotmap.reshape(_B, 1), perm.reshape(_B, 1), subx,
      mu_W1, mu_b1.reshape(1, _H), mu_W2, mu_b2.reshape(1, _D),
      lv_W1, lv_b1.reshape(1, _H), lv_W2, lv_b2.reshape(1, _D))


def kernel(init_embed, pca_W, pca_b, rel, mu_W1, mu_b1, mu_W2, mu_b2,
           lv_W1, lv_b1, lv_W2, lv_b2, edge_index, edge_type, sub):
    xpre = _compute_xpre(init_embed, pca_W, pca_b)

    relx2 = jnp.tile(rel, (1, _F))
    negones = jnp.full((_NPOS,), -1, jnp.int32)
    zeros = jnp.zeros((_AGG_ROWS, _WROW), jnp.float32)

    agg2, slotmap, subx = _run_mp(edge_index, edge_type, xpre, relx2, sub,
                                  negones, zeros)

    perm = jax.random.permutation(jax.random.key(1), _B).astype(jnp.int32)
    out = _run_club(agg2, slotmap, perm, subx,
                    mu_W1, mu_b1, mu_W2, mu_b2, lv_W1, lv_b1, lv_W2, lv_b2)
    return out.reshape(())


# speedup vs baseline: 469.6689x; 1.2359x over previous
import functools

import jax
import jax.numpy as jnp
from jax import lax
from jax.experimental import pallas as pl
from jax.experimental.pallas import tpu as pltpu
from jax.experimental.pallas import tpu_sc as plsc

_N = 50000
_E = 800000
_INIT_DIM = 128
_D = 32
_F = 2
_NREL = 100
_B = 1024
_H = 16
_FD = _F * _D

_NC, _NS, _L = 2, 16, 16
_NW = _NC * _NS

_BLK = 4096
_CHUNK = _E // _NW
_NFULL = _CHUNK // _BLK
_TAIL = _CHUNK - _NFULL * _BLK
_BATCH = 128
_WROW = 80
_NPOS = _N + _L
_AGG_ROWS = _B + 16
_STRIDE = 13 * 2048
_TRASH = 1024 * 256


def _xpre_body(x_ref, w_ref, b_ref, o_ref):
    acc = jnp.dot(x_ref[...], w_ref[...], preferred_element_type=jnp.float32,
                  precision=lax.Precision.DEFAULT)
    o_ref[...] = jnp.tanh(acc + b_ref[...])


def _compute_xpre(init_embed, pca_W, pca_b):
    blk = 2000
    grid = _N // blk
    return pl.pallas_call(
        _xpre_body,
        grid=(grid,),
        in_specs=[
            pl.BlockSpec((blk, _INIT_DIM), lambda i: (i, 0)),
            pl.BlockSpec((_INIT_DIM, _FD), lambda i: (0, 0)),
            pl.BlockSpec((1, _FD), lambda i: (0, 0)),
        ],
        out_specs=pl.BlockSpec((blk, _FD), lambda i: (i, 0)),
        out_shape=jax.ShapeDtypeStruct((_N, _FD), jnp.float32),
    )(init_embed, pca_W, pca_b.reshape(1, _FD))


def _filter_body(ei_hbm, et_hbm, sub_hbm, negones_hbm,
                 srcf_out, psef_out, cnt_out, slotmap_out,
                 pos_v, sub_v, dstb_v, srcb_v, etb_v, srcf_v, psef_v,
                 cntv_v, slotc_v):
    cid = lax.axis_index("c")
    sid = lax.axis_index("s")
    wid = sid * _NC + cid
    base = wid * _CHUNK
    iota = lax.iota(jnp.int32, _L)

    pltpu.sync_copy(sub_hbm, sub_v)
    pltpu.sync_copy(negones_hbm, pos_v)

    def _scatter_pos(i, c):
        nodes = sub_v[pl.ds(i * _L, _L)]
        plsc.store_scatter(pos_v, [nodes], i * _L + iota)
        return c
    lax.fori_loop(0, _B // _L, _scatter_pos, jnp.int32(0))

    def _do_block(off_words, thr, cnt):
        pltpu.sync_copy(ei_hbm.at[1, pl.ds(off_words, _BLK)], dstb_v)
        pltpu.sync_copy(ei_hbm.at[0, pl.ds(off_words, _BLK)], srcb_v)
        pltpu.sync_copy(et_hbm.at[pl.ds(off_words, _BLK)], etb_v)

        def _vec(i, cnt):
            d = dstb_v[pl.ds(i * _L, _L)]
            p = plsc.load_gather(pos_v, [d])
            if thr:
                m = jnp.logical_and(p >= 0, i * _L + iota >= thr)
            else:
                m = p >= 0
            s = srcb_v[pl.ds(i * _L, _L)]
            t = etb_v[pl.ds(i * _L, _L)]
            pse = p * 256 + t
            plsc.store_compressed(srcf_v.at[pl.ds(cnt, _L)], s, mask=m)
            plsc.store_compressed(psef_v.at[pl.ds(cnt, _L)], pse, mask=m)
            return cnt + plsc.all_reduce_population_count(m)[0]
        return lax.fori_loop(0, _BLK // _L, _vec, cnt, unroll=4)

    def _full_block(bi, cnt):
        return _do_block(base + bi * _BLK, 0, cnt)
    cnt = lax.fori_loop(0, _NFULL, _full_block, jnp.int32(0))
    cnt = _do_block(base + _CHUNK - _BLK, _BLK - _TAIL, cnt)

    ones = jnp.full((_L,), True)

    def _pad(i, c):
        plsc.store_compressed(srcf_v.at[pl.ds(cnt + i * _L, _L)],
                              jnp.zeros((_L,), jnp.int32), mask=ones)
        plsc.store_compressed(psef_v.at[pl.ds(cnt + i * _L, _L)],
                              jnp.full((_L,), _TRASH, jnp.int32), mask=ones)
        return c
    lax.fori_loop(0, _BATCH // _L, _pad, jnp.int32(0))

    hb = wid * _STRIDE
    nblk = lax.div(cnt + jnp.int32(_BATCH + 2047), jnp.int32(2048))

    def _out(b, c):
        pltpu.sync_copy(srcf_v.at[pl.ds(b * 2048, 2048)],
                        srcf_out.at[pl.ds(hb + b * 2048, 2048)])
        pltpu.sync_copy(psef_v.at[pl.ds(b * 2048, 2048)],
                        psef_out.at[pl.ds(hb + b * 2048, 2048)])
        return c
    lax.fori_loop(0, nblk, _out, jnp.int32(0))

    cntv_v[pl.ds(0, _L)] = jnp.broadcast_to(cnt, (_L,)).astype(jnp.int32)
    pltpu.sync_copy(cntv_v, cnt_out.at[pl.ds(wid * _L, _L)])

    rbase = wid * 32
    s0 = sub_v[pl.ds(rbase, _L)]
    s1 = sub_v[pl.ds(rbase + _L, _L)]
    slotc_v[pl.ds(0, _L)] = plsc.load_gather(pos_v, [s0])
    slotc_v[pl.ds(_L, _L)] = plsc.load_gather(pos_v, [s1])
    pltpu.sync_copy(slotc_v, slotmap_out.at[pl.ds(rbase, 32)])


def _run_filter(edge_index, edge_type, sub, negones):
    mesh = plsc.VectorSubcoreMesh(
        core_axis_name="c", subcore_axis_name="s",
        num_cores=_NC, num_subcores=_NS)
    f = functools.partial(
        pl.kernel,
        out_type=[
            jax.ShapeDtypeStruct((_NW * _STRIDE,), jnp.int32),
            jax.ShapeDtypeStruct((_NW * _STRIDE,), jnp.int32),
            jax.ShapeDtypeStruct((_NW * _L,), jnp.int32),
            jax.ShapeDtypeStruct((_B,), jnp.int32),
        ],
        mesh=mesh,
        compiler_params=pltpu.CompilerParams(
            needs_layout_passes=False, use_tc_tiling_on_sc=False),
        scratch_types=[
            pltpu.VMEM((_NPOS,), jnp.int32),
            pltpu.VMEM((_B,), jnp.int32),
            pltpu.VMEM((_BLK,), jnp.int32),
            pltpu.VMEM((_BLK,), jnp.int32),
            pltpu.VMEM((_BLK,), jnp.int32),
            pltpu.VMEM((_STRIDE,), jnp.int32),
            pltpu.VMEM((_STRIDE,), jnp.int32),
            pltpu.VMEM((_L,), jnp.int32),
            pltpu.VMEM((32,), jnp.int32),
        ],
    )(_filter_body)
    return f(edge_index, edge_type, sub, negones)


def _agg_body(srcf_hbm, psef_hbm, cnt_hbm, xpre_hbm, relx2_hbm, sub_hbm,
              zeros_hbm,
              agg_out, subx_out,
              srcv_v, psev_v, slot_v, etv_v, xrows_v, relrows_v, msg_v,
              cntv_v, sub32_v, subxc_v, agg_sh):
    cid = lax.axis_index("c")
    sid = lax.axis_index("s")
    wid = sid * _NC + cid
    iota = lax.iota(jnp.int32, _L)

    @pl.when(sid == 0)
    def _zero():
        pltpu.sync_copy(zeros_hbm, agg_sh)

    deg_vec = jnp.where(iota == 0, 1.0, 0.0).astype(jnp.float32)

    def _deg_init(e, c):
        msg_v[e, pl.ds(_FD, _L)] = deg_vec
        return c
    lax.fori_loop(0, _BATCH, _deg_init, jnp.int32(0))

    pltpu.sync_copy(cnt_hbm.at[pl.ds(wid * _L, _L)], cntv_v)

    plsc.subcore_barrier()

    cnt = cntv_v[pl.ds(0, _L)][0]
    hb = wid * _STRIDE
    nb = lax.div(cnt + jnp.int32(_BATCH - 1), jnp.int32(_BATCH))

    def _batch(bi, c):
        off = hb + bi * _BATCH
        pltpu.sync_copy(srcf_hbm.at[pl.ds(off, _BATCH)], srcv_v)
        pltpu.sync_copy(psef_hbm.at[pl.ds(off, _BATCH)], psev_v)

        def _unp(i, c2):
            v = psev_v[pl.ds(i * _L, _L)]
            slot_v[pl.ds(i * _L, _L)] = lax.shift_right_logical(v, 8)
            etv_v[pl.ds(i * _L, _L)] = lax.bitwise_and(v, jnp.int32(255))
            return c2
        lax.fori_loop(0, _BATCH // _L, _unp, jnp.int32(0))

        pltpu.sync_copy(xpre_hbm.at[srcv_v], xrows_v)
        pltpu.sync_copy(relx2_hbm.at[etv_v], relrows_v)

        def _mul(e, c2):
            for j in range(_FD // _L):
                a = xrows_v[e, pl.ds(j * _L, _L)]
                r = relrows_v[e, pl.ds(j * _L, _L)]
                msg_v[e, pl.ds(j * _L, _L)] = a * r
            return c2
        lax.fori_loop(0, _BATCH, _mul, jnp.int32(0), unroll=2)

        pltpu.sync_copy(msg_v, agg_sh.at[slot_v], add=True)
        return c
    lax.fori_loop(0, nb, _batch, jnp.int32(0))

    plsc.subcore_barrier()

    @pl.when(sid == 0)
    def _flush():
        pltpu.sync_copy(agg_sh, agg_out.at[cid])

    rbase = wid * 32
    pltpu.sync_copy(sub_hbm.at[pl.ds(rbase, 32)], sub32_v)
    pltpu.sync_copy(xpre_hbm.at[sub32_v], subxc_v)
    pltpu.sync_copy(subxc_v, subx_out.at[pl.ds(rbase, 32)])


def _run_agg(srcf, psef, cnts, xpre, relx2, sub, zeros):
    mesh = plsc.VectorSubcoreMesh(
        core_axis_name="c", subcore_axis_name="s",
        num_cores=_NC, num_subcores=_NS)
    f = functools.partial(
        pl.kernel,
        out_type=[
            jax.ShapeDtypeStruct((_NC, _AGG_ROWS, _WROW), jnp.float32),
            jax.ShapeDtypeStruct((_B, _FD), jnp.float32),
        ],
        mesh=mesh,
        compiler_params=pltpu.CompilerParams(
            needs_layout_passes=False, use_tc_tiling_on_sc=False),
        scratch_types=[
            pltpu.VMEM((_BATCH,), jnp.int32),
            pltpu.VMEM((_BATCH,), jnp.int32),
            pltpu.VMEM((_BATCH,), jnp.int32),
            pltpu.VMEM((_BATCH,), jnp.int32),
            pltpu.VMEM((_BATCH, _FD), jnp.float32),
            pltpu.VMEM((_BATCH, _FD), jnp.float32),
            pltpu.VMEM((_BATCH, _WROW), jnp.float32),
            pltpu.VMEM((_L,), jnp.int32),
            pltpu.VMEM((32,), jnp.int32),
            pltpu.VMEM((32, _FD), jnp.float32),
            pltpu.VMEM_SHARED((_AGG_ROWS, _WROW), jnp.float32),
        ],
    )(_agg_body)
    return f(srcf, psef, cnts, xpre, relx2, sub, zeros)


def _club_body(agg2_ref, slot_ref, perm_ref, subx_ref,
               muW1, mub1, muW2, mub2, lvW1, lvb1, lvW2, lvb2, o_ref):
    agg = agg2_ref[0] + agg2_ref[1]
    oh = (slot_ref[...] == lax.broadcasted_iota(
        jnp.int32, (_B, _AGG_ROWS), 1)).astype(jnp.float32)
    sel = jnp.dot(oh, agg, preferred_element_type=jnp.float32,
                  precision=lax.Precision.HIGHEST)
    msg = sel[:, :_FD]
    deg = jnp.maximum(sel[:, _FD:_FD + 1], 1.0)
    xsub = jnp.tanh(subx_ref[...] + msg / deg)
    xs = xsub[:, :_D]
    ys = xsub[:, _D:_FD]
    hmu = jnp.maximum(jnp.dot(xs, muW1[...], preferred_element_type=jnp.float32,
                              precision=lax.Precision.DEFAULT)
                      + mub1[...], 0.0)
    mu = jnp.dot(hmu, muW2[...], preferred_element_type=jnp.float32,
                 precision=lax.Precision.DEFAULT) + mub2[...]
    hlv = jnp.maximum(jnp.dot(xs, lvW1[...], preferred_element_type=jnp.float32,
                              precision=lax.Precision.DEFAULT)
                      + lvb1[...], 0.0)
    logvar = jnp.tanh(jnp.dot(hlv, lvW2[...], preferred_element_type=jnp.float32,
                              precision=lax.Precision.DEFAULT)
                      + lvb2[...])
    inv = jnp.exp(-logvar)
    ohp = (perm_ref[...] == lax.broadcasted_iota(
        jnp.int32, (_B, _B), 1)).astype(jnp.float32)
    ysp = jnp.dot(ohp, ys, preferred_element_type=jnp.float32,
                  precision=lax.Precision.HIGHEST)
    pos_t = ((mu - ys) ** 2) * inv
    neg_t = ((mu - ysp) ** 2) * inv
    val = (jnp.sum(neg_t) - jnp.sum(pos_t)) / (2.0 * _B)
    o_ref[...] = val.reshape(1, 1)


def _run_club(agg2, slotmap, perm, subx,
              mu_W1, mu_b1, mu_W2, mu_b2, lv_W1, lv_b1, lv_W2, lv_b2):
    return pl.pallas_call(
        _club_body,
        out_shape=jax.ShapeDtypeStruct((1, 1), jnp.float32),
    )(agg2, slotmap.reshape(_B, 1), perm.reshape(_B, 1), subx,
      mu_W1, mu_b1.reshape(1, _H), mu_W2, mu_b2.reshape(1, _D),
      lv_W1, lv_b1.reshape(1, _H), lv_W2, lv_b2.reshape(1, _D))


def kernel(init_embed, pca_W, pca_b, rel, mu_W1, mu_b1, mu_W2, mu_b2,
           lv_W1, lv_b1, lv_W2, lv_b2, edge_index, edge_type, sub):
    negones = jnp.full((_NPOS,), -1, jnp.int32)
    zeros = jnp.zeros((_AGG_ROWS, _WROW), jnp.float32)
    relx2 = jnp.tile(rel, (1, _F))

    srcf, psef, cnts, slotmap = _run_filter(edge_index, edge_type, sub,
                                            negones)
    xpre = _compute_xpre(init_embed, pca_W, pca_b)
    agg2, subx = _run_agg(srcf, psef, cnts, xpre, relx2, sub, zeros)

    perm = jax.random.permutation(jax.random.key(1), _B).astype(jnp.int32)
    out = _run_club(agg2, slotmap, perm, subx,
                    mu_W1, mu_b1, mu_W2, mu_b2, lv_W1, lv_b1, lv_W2, lv_b2)
    return out.reshape(())


# speedup vs baseline: 535.6025x; 1.1404x over previous
import functools

import jax
import jax.numpy as jnp
from jax import lax
from jax.experimental import pallas as pl
from jax.experimental.pallas import tpu as pltpu
from jax.experimental.pallas import tpu_sc as plsc

_N = 50000
_E = 800000
_INIT_DIM = 128
_D = 32
_F = 2
_NREL = 100
_B = 1024
_H = 16
_FD = _F * _D

_NC, _NS, _L = 2, 16, 16
_NW = _NC * _NS

_BLK = 4096
_CHUNK = _E // _NW
_NFULL = _CHUNK // _BLK
_TAIL = _CHUNK - _NFULL * _BLK
_BATCH = 128
_WROW = 80
_NPOS = _N + _L
_AGG_ROWS = _B + 16
_STRIDE = 13 * 2048
_TRASH = 1024 * 256


def _xpre_body(x_ref, w_ref, b_ref, o_ref):
    acc = jnp.dot(x_ref[...], w_ref[...], preferred_element_type=jnp.float32,
                  precision=lax.Precision.DEFAULT)
    o_ref[...] = jnp.tanh(acc + b_ref[...])


def _compute_xpre(init_embed, pca_W, pca_b):
    blk = 2000
    grid = _N // blk
    return pl.pallas_call(
        _xpre_body,
        grid=(grid,),
        in_specs=[
            pl.BlockSpec((blk, _INIT_DIM), lambda i: (i, 0)),
            pl.BlockSpec((_INIT_DIM, _FD), lambda i: (0, 0)),
            pl.BlockSpec((1, _FD), lambda i: (0, 0)),
        ],
        out_specs=pl.BlockSpec((blk, _FD), lambda i: (i, 0)),
        out_shape=jax.ShapeDtypeStruct((_N, _FD), jnp.float32),
    )(init_embed, pca_W, pca_b.reshape(1, _FD))


def _filter_body(ei_hbm, et_hbm, sub_hbm, negones_hbm,
                 srcf_out, psef_out, cnt_out, slotmap_out,
                 pos_v, sub_v, dstb_v, srcb_v, etb_v, srcf_v, psef_v,
                 cntv_v, slotc_v, fsem1, fsem2, fsem3):
    cid = lax.axis_index("c")
    sid = lax.axis_index("s")
    wid = sid * _NC + cid
    base = wid * _CHUNK
    iota = lax.iota(jnp.int32, _L)

    pltpu.sync_copy(sub_hbm, sub_v)
    pltpu.sync_copy(negones_hbm, pos_v)

    def _scatter_pos(i, c):
        nodes = sub_v[pl.ds(i * _L, _L)]
        plsc.store_scatter(pos_v, [nodes], i * _L + iota)
        return c
    lax.fori_loop(0, _B // _L, _scatter_pos, jnp.int32(0))

    def _do_block(off_words, thr, cnt):
        c1 = pltpu.async_copy(ei_hbm.at[1, pl.ds(off_words, _BLK)], dstb_v,
                              fsem1)
        c2 = pltpu.async_copy(ei_hbm.at[0, pl.ds(off_words, _BLK)], srcb_v,
                              fsem2)
        c3 = pltpu.async_copy(et_hbm.at[pl.ds(off_words, _BLK)], etb_v, fsem3)
        c1.wait()
        c2.wait()
        c3.wait()

        def _vec(i, cnt):
            d = dstb_v[pl.ds(i * _L, _L)]
            p = plsc.load_gather(pos_v, [d])
            if thr:
                m = jnp.logical_and(p >= 0, i * _L + iota >= thr)
            else:
                m = p >= 0
            s = srcb_v[pl.ds(i * _L, _L)]
            t = etb_v[pl.ds(i * _L, _L)]
            pse = p * 256 + t
            plsc.store_compressed(srcf_v.at[pl.ds(cnt, _L)], s, mask=m)
            plsc.store_compressed(psef_v.at[pl.ds(cnt, _L)], pse, mask=m)
            return cnt + plsc.all_reduce_population_count(m)[0]
        return lax.fori_loop(0, _BLK // _L, _vec, cnt, unroll=4)

    def _full_block(bi, cnt):
        return _do_block(base + bi * _BLK, 0, cnt)
    cnt = lax.fori_loop(0, _NFULL, _full_block, jnp.int32(0))
    cnt = _do_block(base + _CHUNK - _BLK, _BLK - _TAIL, cnt)

    ones = jnp.full((_L,), True)

    def _pad(i, c):
        plsc.store_compressed(srcf_v.at[pl.ds(cnt + i * _L, _L)],
                              jnp.zeros((_L,), jnp.int32), mask=ones)
        plsc.store_compressed(psef_v.at[pl.ds(cnt + i * _L, _L)],
                              jnp.full((_L,), _TRASH, jnp.int32), mask=ones)
        return c
    lax.fori_loop(0, _BATCH // _L, _pad, jnp.int32(0))

    hb = wid * _STRIDE
    nblk = lax.div(cnt + jnp.int32(_BATCH + 2047), jnp.int32(2048))

    def _out(b, c):
        pltpu.sync_copy(srcf_v.at[pl.ds(b * 2048, 2048)],
                        srcf_out.at[pl.ds(hb + b * 2048, 2048)])
        pltpu.sync_copy(psef_v.at[pl.ds(b * 2048, 2048)],
                        psef_out.at[pl.ds(hb + b * 2048, 2048)])
        return c
    lax.fori_loop(0, nblk, _out, jnp.int32(0))

    cntv_v[pl.ds(0, _L)] = jnp.broadcast_to(cnt, (_L,)).astype(jnp.int32)
    pltpu.sync_copy(cntv_v, cnt_out.at[pl.ds(wid * _L, _L)])

    rbase = wid * 32
    s0 = sub_v[pl.ds(rbase, _L)]
    s1 = sub_v[pl.ds(rbase + _L, _L)]
    slotc_v[pl.ds(0, _L)] = plsc.load_gather(pos_v, [s0])
    slotc_v[pl.ds(_L, _L)] = plsc.load_gather(pos_v, [s1])
    pltpu.sync_copy(slotc_v, slotmap_out.at[pl.ds(rbase, 32)])


def _run_filter(edge_index, edge_type, sub, negones):
    mesh = plsc.VectorSubcoreMesh(
        core_axis_name="c", subcore_axis_name="s",
        num_cores=_NC, num_subcores=_NS)
    f = functools.partial(
        pl.kernel,
        out_type=[
            jax.ShapeDtypeStruct((_NW * _STRIDE,), jnp.int32),
            jax.ShapeDtypeStruct((_NW * _STRIDE,), jnp.int32),
            jax.ShapeDtypeStruct((_NW * _L,), jnp.int32),
            jax.ShapeDtypeStruct((_B,), jnp.int32),
        ],
        mesh=mesh,
        compiler_params=pltpu.CompilerParams(
            needs_layout_passes=False, use_tc_tiling_on_sc=False),
        scratch_types=[
            pltpu.VMEM((_NPOS,), jnp.int32),
            pltpu.VMEM((_B,), jnp.int32),
            pltpu.VMEM((_BLK,), jnp.int32),
            pltpu.VMEM((_BLK,), jnp.int32),
            pltpu.VMEM((_BLK,), jnp.int32),
            pltpu.VMEM((_STRIDE,), jnp.int32),
            pltpu.VMEM((_STRIDE,), jnp.int32),
            pltpu.VMEM((_L,), jnp.int32),
            pltpu.VMEM((32,), jnp.int32),
            pltpu.SemaphoreType.DMA,
            pltpu.SemaphoreType.DMA,
            pltpu.SemaphoreType.DMA,
        ],
    )(_filter_body)
    return f(edge_index, edge_type, sub, negones)


def _agg_body(srcf_hbm, psef_hbm, cnt_hbm, xpre_hbm, relx2_hbm, sub_hbm,
              zeros_hbm,
              agg_out, subx_out,
              srcall_v, psall_v, slot_v, etv_v, xrows_v, relrows_v, msg_v,
              cntv_v, sub32_v, subxc_v, agg_sh, gsem1, gsem2):
    cid = lax.axis_index("c")
    sid = lax.axis_index("s")
    wid = sid * _NC + cid
    iota = lax.iota(jnp.int32, _L)

    @pl.when(sid == 0)
    def _zero():
        pltpu.sync_copy(zeros_hbm, agg_sh)

    deg_vec = jnp.where(iota == 0, 1.0, 0.0).astype(jnp.float32)

    def _deg_init(e, c):
        msg_v[e, pl.ds(_FD, _L)] = deg_vec
        return c
    lax.fori_loop(0, _BATCH, _deg_init, jnp.int32(0))

    pltpu.sync_copy(cnt_hbm.at[pl.ds(wid * _L, _L)], cntv_v)

    plsc.subcore_barrier()

    cnt = cntv_v[pl.ds(0, _L)][0]
    hb = wid * _STRIDE
    nb = lax.div(cnt + jnp.int32(_BATCH - 1), jnp.int32(_BATCH))

    nblk = lax.div(cnt + jnp.int32(_BATCH + 2047), jnp.int32(2048))

    def _lst(b, c):
        c1 = pltpu.async_copy(srcf_hbm.at[pl.ds(hb + b * 2048, 2048)],
                              srcall_v.at[pl.ds(b * 2048, 2048)], gsem1)
        c2 = pltpu.async_copy(psef_hbm.at[pl.ds(hb + b * 2048, 2048)],
                              psall_v.at[pl.ds(b * 2048, 2048)], gsem2)
        c1.wait()
        c2.wait()
        return c
    lax.fori_loop(0, nblk, _lst, jnp.int32(0))

    def _batch(bi, c):
        loff = bi * _BATCH

        def _unp(i, c2):
            v = psall_v[pl.ds(loff + i * _L, _L)]
            slot_v[pl.ds(i * _L, _L)] = lax.shift_right_logical(v, 8)
            etv_v[pl.ds(i * _L, _L)] = lax.bitwise_and(v, jnp.int32(255))
            return c2
        lax.fori_loop(0, _BATCH // _L, _unp, jnp.int32(0))

        g1 = pltpu.async_copy(xpre_hbm.at[srcall_v.at[pl.ds(loff, _BATCH)]],
                              xrows_v, gsem1)
        g2 = pltpu.async_copy(relx2_hbm.at[etv_v], relrows_v, gsem2)
        g1.wait()
        g2.wait()

        def _mul(e, c2):
            for j in range(_FD // _L):
                a = xrows_v[e, pl.ds(j * _L, _L)]
                r = relrows_v[e, pl.ds(j * _L, _L)]
                msg_v[e, pl.ds(j * _L, _L)] = a * r
            return c2
        lax.fori_loop(0, _BATCH, _mul, jnp.int32(0), unroll=2)

        pltpu.sync_copy(msg_v, agg_sh.at[slot_v], add=True)
        return c
    lax.fori_loop(0, nb, _batch, jnp.int32(0))

    plsc.subcore_barrier()

    @pl.when(sid == 0)
    def _flush():
        pltpu.sync_copy(agg_sh, agg_out.at[cid])

    rbase = wid * 32
    pltpu.sync_copy(sub_hbm.at[pl.ds(rbase, 32)], sub32_v)
    pltpu.sync_copy(xpre_hbm.at[sub32_v], subxc_v)
    pltpu.sync_copy(subxc_v, subx_out.at[pl.ds(rbase, 32)])


def _run_agg(srcf, psef, cnts, xpre, relx2, sub, zeros):
    mesh = plsc.VectorSubcoreMesh(
        core_axis_name="c", subcore_axis_name="s",
        num_cores=_NC, num_subcores=_NS)
    f = functools.partial(
        pl.kernel,
        out_type=[
            jax.ShapeDtypeStruct((_NC, _AGG_ROWS, _WROW), jnp.float32),
            jax.ShapeDtypeStruct((_B, _FD), jnp.float32),
        ],
        mesh=mesh,
        compiler_params=pltpu.CompilerParams(
            needs_layout_passes=False, use_tc_tiling_on_sc=False),
        scratch_types=[
            pltpu.VMEM((_STRIDE,), jnp.int32),
            pltpu.VMEM((_STRIDE,), jnp.int32),
            pltpu.VMEM((_BATCH,), jnp.int32),
            pltpu.VMEM((_BATCH,), jnp.int32),
            pltpu.VMEM((_BATCH, _FD), jnp.float32),
            pltpu.VMEM((_BATCH, _FD), jnp.float32),
            pltpu.VMEM((_BATCH, _WROW), jnp.float32),
            pltpu.VMEM((_L,), jnp.int32),
            pltpu.VMEM((32,), jnp.int32),
            pltpu.VMEM((32, _FD), jnp.float32),
            pltpu.VMEM_SHARED((_AGG_ROWS, _WROW), jnp.float32),
            pltpu.SemaphoreType.DMA,
            pltpu.SemaphoreType.DMA,
        ],
    )(_agg_body)
    return f(srcf, psef, cnts, xpre, relx2, sub, zeros)


def _club_body(agg2_ref, slot_ref, perm_ref, subx_ref,
               muW1, mub1, muW2, mub2, lvW1, lvb1, lvW2, lvb2, o_ref):
    agg = agg2_ref[0] + agg2_ref[1]
    oh = (slot_ref[...] == lax.broadcasted_iota(
        jnp.int32, (_B, _AGG_ROWS), 1)).astype(jnp.float32)
    sel = jnp.dot(oh, agg, preferred_element_type=jnp.float32,
                  precision=lax.Precision.HIGHEST)
    msg = sel[:, :_FD]
    deg = jnp.maximum(sel[:, _FD:_FD + 1], 1.0)
    xsub = jnp.tanh(subx_ref[...] + msg / deg)
    xs = xsub[:, :_D]
    ys = xsub[:, _D:_FD]
    hmu = jnp.maximum(jnp.dot(xs, muW1[...], preferred_element_type=jnp.float32,
                              precision=lax.Precision.DEFAULT)
                      + mub1[...], 0.0)
    mu = jnp.dot(hmu, muW2[...], preferred_element_type=jnp.float32,
                 precision=lax.Precision.DEFAULT) + mub2[...]
    hlv = jnp.maximum(jnp.dot(xs, lvW1[...], preferred_element_type=jnp.float32,
                              precision=lax.Precision.DEFAULT)
                      + lvb1[...], 0.0)
    logvar = jnp.tanh(jnp.dot(hlv, lvW2[...], preferred_element_type=jnp.float32,
                              precision=lax.Precision.DEFAULT)
                      + lvb2[...])
    inv = jnp.exp(-logvar)
    ohp = (perm_ref[...] == lax.broadcasted_iota(
        jnp.int32, (_B, _B), 1)).astype(jnp.float32)
    ysp = jnp.dot(ohp, ys, preferred_element_type=jnp.float32,
                  precision=lax.Precision.HIGHEST)
    pos_t = ((mu - ys) ** 2) * inv
    neg_t = ((mu - ysp) ** 2) * inv
    val = (jnp.sum(neg_t) - jnp.sum(pos_t)) / (2.0 * _B)
    o_ref[...] = val.reshape(1, 1)


def _run_club(agg2, slotmap, perm, subx,
              mu_W1, mu_b1, mu_W2, mu_b2, lv_W1, lv_b1, lv_W2, lv_b2):
    return pl.pallas_call(
        _club_body,
        out_shape=jax.ShapeDtypeStruct((1, 1), jnp.float32),
    )(agg2, slotmap.reshape(_B, 1), perm.reshape(_B, 1), subx,
      mu_W1, mu_b1.reshape(1, _H), mu_W2, mu_b2.reshape(1, _D),
      lv_W1, lv_b1.reshape(1, _H), lv_W2, lv_b2.reshape(1, _D))


def kernel(init_embed, pca_W, pca_b, rel, mu_W1, mu_b1, mu_W2, mu_b2,
           lv_W1, lv_b1, lv_W2, lv_b2, edge_index, edge_type, sub):
    negones = jnp.full((_NPOS,), -1, jnp.int32)
    zeros = jnp.zeros((_AGG_ROWS, _WROW), jnp.float32)
    relx2 = jnp.tile(rel, (1, _F))

    srcf, psef, cnts, slotmap = _run_filter(edge_index, edge_type, sub,
                                            negones)
    xpre = _compute_xpre(init_embed, pca_W, pca_b)
    agg2, subx = _run_agg(srcf, psef, cnts, xpre, relx2, sub, zeros)

    perm = jax.random.permutation(jax.random.key(1), _B).astype(jnp.int32)
    out = _run_club(agg2, slotmap, perm, subx,
                    mu_W1, mu_b1, mu_W2, mu_b2, lv_W1, lv_b1, lv_W2, lv_b2)
    return out.reshape(())


# speedup vs baseline: 575.3448x; 1.0742x over previous
import functools

import jax
import jax.numpy as jnp
from jax import lax
from jax.experimental import pallas as pl
from jax.experimental.pallas import tpu as pltpu
from jax.experimental.pallas import tpu_sc as plsc

_N = 50000
_E = 800000
_INIT_DIM = 128
_D = 32
_F = 2
_NREL = 100
_B = 1024
_H = 16
_FD = _F * _D

_NC, _NS, _L = 2, 16, 16
_NW = _NC * _NS

_BLK = 4096
_CHUNK = _E // _NW
_NFULL = _CHUNK // _BLK
_TAIL = _CHUNK - _NFULL * _BLK
_BATCH = 128
_WROW = 80
_NPOS = _N + _L
_AGG_ROWS = _B + 16
_STRIDE = 13 * 2048
_TRASH = 1024 * 256


def _xpre_body(x_ref, w_ref, b_ref, o_ref):
    acc = jnp.dot(x_ref[...], w_ref[...], preferred_element_type=jnp.float32,
                  precision=lax.Precision.DEFAULT)
    o_ref[...] = jnp.tanh(acc + b_ref[...])


def _compute_xpre(init_embed, pca_W, pca_b):
    blk = 5000
    grid = _N // blk
    return pl.pallas_call(
        _xpre_body,
        grid=(grid,),
        in_specs=[
            pl.BlockSpec((blk, _INIT_DIM), lambda i: (i, 0)),
            pl.BlockSpec((_INIT_DIM, _FD), lambda i: (0, 0)),
            pl.BlockSpec((1, _FD), lambda i: (0, 0)),
        ],
        out_specs=pl.BlockSpec((blk, _FD), lambda i: (i, 0)),
        out_shape=jax.ShapeDtypeStruct((_N, _FD), jnp.float32),
    )(init_embed, pca_W, pca_b.reshape(1, _FD))


def _filter_body(ei_hbm, et_hbm, sub_hbm, negones_hbm,
                 srcf_out, psef_out, cnt_out, slotmap_out,
                 pos_v, sub_v, dstb_v, srcb_v, etb_v, srcf_v, psef_v,
                 cntv_v, slotc_v, fsem1, fsem2, fsem3):
    cid = lax.axis_index("c")
    sid = lax.axis_index("s")
    wid = sid * _NC + cid
    base = wid * _CHUNK
    iota = lax.iota(jnp.int32, _L)

    pltpu.sync_copy(sub_hbm, sub_v)
    pltpu.sync_copy(negones_hbm, pos_v)

    def _scatter_pos(i, c):
        nodes = sub_v[pl.ds(i * _L, _L)]
        plsc.store_scatter(pos_v, [nodes], i * _L + iota)
        return c
    lax.fori_loop(0, _B // _L, _scatter_pos, jnp.int32(0))

    def _do_block(off_words, thr, cnt):
        c1 = pltpu.async_copy(ei_hbm.at[1, pl.ds(off_words, _BLK)], dstb_v,
                              fsem1)
        c2 = pltpu.async_copy(ei_hbm.at[0, pl.ds(off_words, _BLK)], srcb_v,
                              fsem2)
        c3 = pltpu.async_copy(et_hbm.at[pl.ds(off_words, _BLK)], etb_v, fsem3)
        c1.wait()
        c2.wait()
        c3.wait()

        def _vec(i, cnt):
            d = dstb_v[pl.ds(i * _L, _L)]
            p = plsc.load_gather(pos_v, [d])
            if thr:
                m = jnp.logical_and(p >= 0, i * _L + iota >= thr)
            else:
                m = p >= 0
            s = srcb_v[pl.ds(i * _L, _L)]
            t = etb_v[pl.ds(i * _L, _L)]
            pse = p * 256 + t
            plsc.store_compressed(srcf_v.at[pl.ds(cnt, _L)], s, mask=m)
            plsc.store_compressed(psef_v.at[pl.ds(cnt, _L)], pse, mask=m)
            return cnt + plsc.all_reduce_population_count(m)[0]
        return lax.fori_loop(0, _BLK // _L, _vec, cnt, unroll=4)

    def _full_block(bi, cnt):
        return _do_block(base + bi * _BLK, 0, cnt)
    cnt = lax.fori_loop(0, _NFULL, _full_block, jnp.int32(0))
    cnt = _do_block(base + _CHUNK - _BLK, _BLK - _TAIL, cnt)

    ones = jnp.full((_L,), True)

    def _pad(i, c):
        plsc.store_compressed(srcf_v.at[pl.ds(cnt + i * _L, _L)],
                              jnp.zeros((_L,), jnp.int32), mask=ones)
        plsc.store_compressed(psef_v.at[pl.ds(cnt + i * _L, _L)],
                              jnp.full((_L,), _TRASH, jnp.int32), mask=ones)
        return c
    lax.fori_loop(0, _BATCH // _L, _pad, jnp.int32(0))

    hb = wid * _STRIDE
    nblk = lax.div(cnt + jnp.int32(_BATCH + 2047), jnp.int32(2048))

    def _out(b, c):
        pltpu.sync_copy(srcf_v.at[pl.ds(b * 2048, 2048)],
                        srcf_out.at[pl.ds(hb + b * 2048, 2048)])
        pltpu.sync_copy(psef_v.at[pl.ds(b * 2048, 2048)],
                        psef_out.at[pl.ds(hb + b * 2048, 2048)])
        return c
    lax.fori_loop(0, nblk, _out, jnp.int32(0))

    cntv_v[pl.ds(0, _L)] = jnp.broadcast_to(cnt, (_L,)).astype(jnp.int32)
    pltpu.sync_copy(cntv_v, cnt_out.at[pl.ds(wid * _L, _L)])

    rbase = wid * 32
    s0 = sub_v[pl.ds(rbase, _L)]
    s1 = sub_v[pl.ds(rbase + _L, _L)]
    slotc_v[pl.ds(0, _L)] = plsc.load_gather(pos_v, [s0])
    slotc_v[pl.ds(_L, _L)] = plsc.load_gather(pos_v, [s1])
    pltpu.sync_copy(slotc_v, slotmap_out.at[pl.ds(rbase, 32)])


def _run_filter(edge_index, edge_type, sub, negones):
    mesh = plsc.VectorSubcoreMesh(
        core_axis_name="c", subcore_axis_name="s",
        num_cores=_NC, num_subcores=_NS)
    f = functools.partial(
        pl.kernel,
        out_type=[
            jax.ShapeDtypeStruct((_NW * _STRIDE,), jnp.int32),
            jax.ShapeDtypeStruct((_NW * _STRIDE,), jnp.int32),
            jax.ShapeDtypeStruct((_NW * _L,), jnp.int32),
            jax.ShapeDtypeStruct((_B,), jnp.int32),
        ],
        mesh=mesh,
        compiler_params=pltpu.CompilerParams(
            needs_layout_passes=False, use_tc_tiling_on_sc=False),
        scratch_types=[
            pltpu.VMEM((_NPOS,), jnp.int32),
            pltpu.VMEM((_B,), jnp.int32),
            pltpu.VMEM((_BLK,), jnp.int32),
            pltpu.VMEM((_BLK,), jnp.int32),
            pltpu.VMEM((_BLK,), jnp.int32),
            pltpu.VMEM((_STRIDE,), jnp.int32),
            pltpu.VMEM((_STRIDE,), jnp.int32),
            pltpu.VMEM((_L,), jnp.int32),
            pltpu.VMEM((32,), jnp.int32),
            pltpu.SemaphoreType.DMA,
            pltpu.SemaphoreType.DMA,
            pltpu.SemaphoreType.DMA,
        ],
    )(_filter_body)
    return f(edge_index, edge_type, sub, negones)


def _agg_body(srcf_hbm, psef_hbm, cnt_hbm, xpre_hbm, relx2_hbm, sub_hbm,
              zeros_hbm,
              agg_out, subx_out,
              srcall_v, psall_v, slot0_v, slot1_v, et0_v, et1_v,
              xrows0_v, xrows1_v, relrows0_v, relrows1_v, msg_v,
              cntv_v, sub32_v, subxc_v, agg_sh, gsem1, gsem2,
              xsem0, xsem1, rsem0, rsem1):
    cid = lax.axis_index("c")
    sid = lax.axis_index("s")
    wid = sid * _NC + cid
    iota = lax.iota(jnp.int32, _L)

    @pl.when(sid == 0)
    def _zero():
        pltpu.sync_copy(zeros_hbm, agg_sh)

    deg_vec = jnp.where(iota == 0, 1.0, 0.0).astype(jnp.float32)

    def _deg_init(e, c):
        msg_v[e, pl.ds(_FD, _L)] = deg_vec
        return c
    lax.fori_loop(0, _BATCH, _deg_init, jnp.int32(0))

    pltpu.sync_copy(cnt_hbm.at[pl.ds(wid * _L, _L)], cntv_v)

    plsc.subcore_barrier()

    cnt = cntv_v[pl.ds(0, _L)][0]
    hb = wid * _STRIDE
    nb = lax.div(cnt + jnp.int32(_BATCH - 1), jnp.int32(_BATCH))

    nblk = lax.div(cnt + jnp.int32(_BATCH + 2047), jnp.int32(2048))

    def _lst(b, c):
        c1 = pltpu.async_copy(srcf_hbm.at[pl.ds(hb + b * 2048, 2048)],
                              srcall_v.at[pl.ds(b * 2048, 2048)], gsem1)
        c2 = pltpu.async_copy(psef_hbm.at[pl.ds(hb + b * 2048, 2048)],
                              psall_v.at[pl.ds(b * 2048, 2048)], gsem2)
        c1.wait()
        c2.wait()
        return c
    lax.fori_loop(0, nblk, _lst, jnp.int32(0))

    slots = (slot0_v, slot1_v)
    ets = (et0_v, et1_v)
    xrows = (xrows0_v, xrows1_v)
    relrows = (relrows0_v, relrows1_v)
    xsems = (xsem0, xsem1)
    rsems = (rsem0, rsem1)

    def _unpack_issue(bi, buf):
        loff = bi * _BATCH

        def _unp(i, c2):
            v = psall_v[pl.ds(loff + i * _L, _L)]
            slots[buf][pl.ds(i * _L, _L)] = lax.shift_right_logical(v, 8)
            ets[buf][pl.ds(i * _L, _L)] = lax.bitwise_and(v, jnp.int32(255))
            return c2
        lax.fori_loop(0, _BATCH // _L, _unp, jnp.int32(0))
        pltpu.async_copy(xpre_hbm.at[srcall_v.at[pl.ds(loff, _BATCH)]],
                         xrows[buf], xsems[buf])
        pltpu.async_copy(relx2_hbm.at[ets[buf]], relrows[buf], rsems[buf])

    @pl.when(nb > 0)
    def _prologue():
        _unpack_issue(jnp.int32(0), 0)

    def _pair(pi, c):
        for b in range(2):
            i = pi * 2 + b
            nxt = 1 - b

            @pl.when(i < nb)
            def _step(i=i, b=b, nxt=nxt):
                @pl.when(i + 1 < nb)
                def _issue_next():
                    _unpack_issue(i + 1, nxt)

                pltpu.make_async_copy(
                    xpre_hbm.at[srcall_v.at[pl.ds(i * _BATCH, _BATCH)]],
                    xrows[b], xsems[b]).wait()
                pltpu.make_async_copy(
                    relx2_hbm.at[ets[b]], relrows[b], rsems[b]).wait()

                def _mul(e, c2):
                    for j in range(_FD // _L):
                        a = xrows[b][e, pl.ds(j * _L, _L)]
                        r = relrows[b][e, pl.ds(j * _L, _L)]
                        msg_v[e, pl.ds(j * _L, _L)] = a * r
                    return c2
                lax.fori_loop(0, _BATCH, _mul, jnp.int32(0), unroll=2)

                pltpu.sync_copy(msg_v, agg_sh.at[slots[b]], add=True)
        return c
    lax.fori_loop(0, lax.div(nb + jnp.int32(1), jnp.int32(2)), _pair,
                  jnp.int32(0))

    plsc.subcore_barrier()

    @pl.when(sid == 0)
    def _flush():
        pltpu.sync_copy(agg_sh, agg_out.at[cid])

    rbase = wid * 32
    pltpu.sync_copy(sub_hbm.at[pl.ds(rbase, 32)], sub32_v)
    pltpu.sync_copy(xpre_hbm.at[sub32_v], subxc_v)
    pltpu.sync_copy(subxc_v, subx_out.at[pl.ds(rbase, 32)])


def _run_agg(srcf, psef, cnts, xpre, relx2, sub, zeros):
    mesh = plsc.VectorSubcoreMesh(
        core_axis_name="c", subcore_axis_name="s",
        num_cores=_NC, num_subcores=_NS)
    f = functools.partial(
        pl.kernel,
        out_type=[
            jax.ShapeDtypeStruct((_NC, _AGG_ROWS, _WROW), jnp.float32),
            jax.ShapeDtypeStruct((_B, _FD), jnp.float32),
        ],
        mesh=mesh,
        compiler_params=pltpu.CompilerParams(
            needs_layout_passes=False, use_tc_tiling_on_sc=False),
        scratch_types=[
            pltpu.VMEM((_STRIDE,), jnp.int32),
            pltpu.VMEM((_STRIDE,), jnp.int32),
            pltpu.VMEM((_BATCH,), jnp.int32),
            pltpu.VMEM((_BATCH,), jnp.int32),
            pltpu.VMEM((_BATCH,), jnp.int32),
            pltpu.VMEM((_BATCH,), jnp.int32),
            pltpu.VMEM((_BATCH, _FD), jnp.float32),
            pltpu.VMEM((_BATCH, _FD), jnp.float32),
            pltpu.VMEM((_BATCH, _FD), jnp.float32),
            pltpu.VMEM((_BATCH, _FD), jnp.float32),
            pltpu.VMEM((_BATCH, _WROW), jnp.float32),
            pltpu.VMEM((_L,), jnp.int32),
            pltpu.VMEM((32,), jnp.int32),
            pltpu.VMEM((32, _FD), jnp.float32),
            pltpu.VMEM_SHARED((_AGG_ROWS, _WROW), jnp.float32),
            pltpu.SemaphoreType.DMA,
            pltpu.SemaphoreType.DMA,
            pltpu.SemaphoreType.DMA,
            pltpu.SemaphoreType.DMA,
            pltpu.SemaphoreType.DMA,
            pltpu.SemaphoreType.DMA,
        ],
    )(_agg_body)
    return f(srcf, psef, cnts, xpre, relx2, sub, zeros)


def _club_body(agg2_ref, slot_ref, perm_ref, subx_ref,
               muW1, mub1, muW2, mub2, lvW1, lvb1, lvW2, lvb2, o_ref):
    agg = agg2_ref[0] + agg2_ref[1]
    oh = (slot_ref[...] == lax.broadcasted_iota(
        jnp.int32, (_B, _AGG_ROWS), 1)).astype(jnp.float32)
    sel = jnp.dot(oh, agg, preferred_element_type=jnp.float32,
                  precision=lax.Precision.HIGHEST)
    msg = sel[:, :_FD]
    deg = jnp.maximum(sel[:, _FD:_FD + 1], 1.0)
    xsub = jnp.tanh(subx_ref[...] + msg / deg)
    xs = xsub[:, :_D]
    ys = xsub[:, _D:_FD]
    hmu = jnp.maximum(jnp.dot(xs, muW1[...], preferred_element_type=jnp.float32,
                              precision=lax.Precision.DEFAULT)
                      + mub1[...], 0.0)
    mu = jnp.dot(hmu, muW2[...], preferred_element_type=jnp.float32,
                 precision=lax.Precision.DEFAULT) + mub2[...]
    hlv = jnp.maximum(jnp.dot(xs, lvW1[...], preferred_element_type=jnp.float32,
                              precision=lax.Precision.DEFAULT)
                      + lvb1[...], 0.0)
    logvar = jnp.tanh(jnp.dot(hlv, lvW2[...], preferred_element_type=jnp.float32,
                              precision=lax.Precision.DEFAULT)
                      + lvb2[...])
    inv = jnp.exp(-logvar)
    ohp = (perm_ref[...] == lax.broadcasted_iota(
        jnp.int32, (_B, _B), 1)).astype(jnp.float32)
    ysp = jnp.dot(ohp, ys, preferred_element_type=jnp.float32,
                  precision=lax.Precision.HIGHEST)
    pos_t = ((mu - ys) ** 2) * inv
    neg_t = ((mu - ysp) ** 2) * inv
    val = (jnp.sum(neg_t) - jnp.sum(pos_t)) / (2.0 * _B)
    o_ref[...] = val.reshape(1, 1)


def _run_club(agg2, slotmap, perm, subx,
              mu_W1, mu_b1, mu_W2, mu_b2, lv_W1, lv_b1, lv_W2, lv_b2):
    return pl.pallas_call(
        _club_body,
        out_shape=jax.ShapeDtypeStruct((1, 1), jnp.float32),
    )(agg2, slotmap.reshape(_B, 1), perm.reshape(_B, 1), subx,
      mu_W1, mu_b1.reshape(1, _H), mu_W2, mu_b2.reshape(1, _D),
      lv_W1, lv_b1.reshape(1, _H), lv_W2, lv_b2.reshape(1, _D))


def kernel(init_embed, pca_W, pca_b, rel, mu_W1, mu_b1, mu_W2, mu_b2,
           lv_W1, lv_b1, lv_W2, lv_b2, edge_index, edge_type, sub):
    negones = jnp.full((_NPOS,), -1, jnp.int32)
    zeros = jnp.zeros((_AGG_ROWS, _WROW), jnp.float32)
    relx2 = jnp.tile(rel, (1, _F))

    srcf, psef, cnts, slotmap = _run_filter(edge_index, edge_type, sub,
                                            negones)
    xpre = _compute_xpre(init_embed, pca_W, pca_b)
    agg2, subx = _run_agg(srcf, psef, cnts, xpre, relx2, sub, zeros)

    perm = jax.random.permutation(jax.random.key(1), _B).astype(jnp.int32)
    out = _run_club(agg2, slotmap, perm, subx,
                    mu_W1, mu_b1, mu_W2, mu_b2, lv_W1, lv_b1, lv_W2, lv_b2)
    return out.reshape(())


# speedup vs baseline: 576.5688x; 1.0021x over previous
import functools

import jax
import jax.numpy as jnp
from jax import lax
from jax.experimental import pallas as pl
from jax.experimental.pallas import tpu as pltpu
from jax.experimental.pallas import tpu_sc as plsc

_N = 50000
_E = 800000
_INIT_DIM = 128
_D = 32
_F = 2
_NREL = 100
_B = 1024
_H = 16
_FD = _F * _D

_NC, _NS, _L = 2, 16, 16
_NW = _NC * _NS

_BLK = 4096
_CHUNK = _E // _NW
_NFULL = _CHUNK // _BLK
_TAIL = _CHUNK - _NFULL * _BLK
_BATCH = 128
_WROW = 80
_NPOS = _N + _L
_AGG_ROWS = _B + 16
_STRIDE = 13 * 2048
_TRASH = 1024 * 256


def _xpre_body(x_ref, w_ref, b_ref, o_ref):
    acc = jnp.dot(x_ref[...], w_ref[...], preferred_element_type=jnp.float32,
                  precision=lax.Precision.DEFAULT)
    o_ref[...] = jnp.tanh(acc + b_ref[...])


def _compute_xpre(init_embed, pca_W, pca_b):
    blk = 5000
    grid = _N // blk
    return pl.pallas_call(
        _xpre_body,
        grid=(grid,),
        in_specs=[
            pl.BlockSpec((blk, _INIT_DIM), lambda i: (i, 0)),
            pl.BlockSpec((_INIT_DIM, _FD), lambda i: (0, 0)),
            pl.BlockSpec((1, _FD), lambda i: (0, 0)),
        ],
        out_specs=pl.BlockSpec((blk, _FD), lambda i: (i, 0)),
        out_shape=jax.ShapeDtypeStruct((_N, _FD), jnp.float32),
    )(init_embed, pca_W, pca_b.reshape(1, _FD))


def _filter_body(ei_hbm, et_hbm, sub_hbm, negones_hbm,
                 srcf_out, psef_out, cnt_out, slotmap_out,
                 pos_v, sub_v, dstb_v, srcb_v, etb_v, srcf_v, psef_v,
                 cntv_v, slotc_v, fsem1, fsem2, fsem3):
    cid = lax.axis_index("c")
    sid = lax.axis_index("s")
    wid = sid * _NC + cid
    base = wid * _CHUNK
    iota = lax.iota(jnp.int32, _L)

    pltpu.sync_copy(sub_hbm, sub_v)
    pltpu.sync_copy(negones_hbm, pos_v)

    def _scatter_pos(i, c):
        nodes = sub_v[pl.ds(i * _L, _L)]
        plsc.store_scatter(pos_v, [nodes], i * _L + iota)
        return c
    lax.fori_loop(0, _B // _L, _scatter_pos, jnp.int32(0))

    def _do_block(off_words, thr, cnt):
        c1 = pltpu.async_copy(ei_hbm.at[1, pl.ds(off_words, _BLK)], dstb_v,
                              fsem1)
        c2 = pltpu.async_copy(ei_hbm.at[0, pl.ds(off_words, _BLK)], srcb_v,
                              fsem2)
        c3 = pltpu.async_copy(et_hbm.at[pl.ds(off_words, _BLK)], etb_v, fsem3)
        c1.wait()
        c2.wait()
        c3.wait()

        def _vec(i, cnt):
            d = dstb_v[pl.ds(i * _L, _L)]
            p = plsc.load_gather(pos_v, [d])
            if thr:
                m = jnp.logical_and(p >= 0, i * _L + iota >= thr)
            else:
                m = p >= 0
            s = srcb_v[pl.ds(i * _L, _L)]
            t = etb_v[pl.ds(i * _L, _L)]
            pse = p * 256 + t
            plsc.store_compressed(srcf_v.at[pl.ds(cnt, _L)], s, mask=m)
            plsc.store_compressed(psef_v.at[pl.ds(cnt, _L)], pse, mask=m)
            return cnt + plsc.all_reduce_population_count(m)[0]
        return lax.fori_loop(0, _BLK // _L, _vec, cnt, unroll=4)

    def _full_block(bi, cnt):
        return _do_block(base + bi * _BLK, 0, cnt)
    cnt = lax.fori_loop(0, _NFULL, _full_block, jnp.int32(0))
    cnt = _do_block(base + _CHUNK - _BLK, _BLK - _TAIL, cnt)

    ones = jnp.full((_L,), True)

    def _pad(i, c):
        plsc.store_compressed(srcf_v.at[pl.ds(cnt + i * _L, _L)],
                              jnp.zeros((_L,), jnp.int32), mask=ones)
        plsc.store_compressed(psef_v.at[pl.ds(cnt + i * _L, _L)],
                              jnp.full((_L,), _TRASH, jnp.int32), mask=ones)
        return c
    lax.fori_loop(0, _BATCH // _L, _pad, jnp.int32(0))

    hb = wid * _STRIDE
    nblk = lax.div(cnt + jnp.int32(_BATCH + 2047), jnp.int32(2048))

    def _out(b, c):
        pltpu.sync_copy(srcf_v.at[pl.ds(b * 2048, 2048)],
                        srcf_out.at[pl.ds(hb + b * 2048, 2048)])
        pltpu.sync_copy(psef_v.at[pl.ds(b * 2048, 2048)],
                        psef_out.at[pl.ds(hb + b * 2048, 2048)])
        return c
    lax.fori_loop(0, nblk, _out, jnp.int32(0))

    cntv_v[pl.ds(0, _L)] = jnp.broadcast_to(cnt, (_L,)).astype(jnp.int32)
    pltpu.sync_copy(cntv_v, cnt_out.at[pl.ds(wid * _L, _L)])

    rbase = wid * 32
    s0 = sub_v[pl.ds(rbase, _L)]
    s1 = sub_v[pl.ds(rbase + _L, _L)]
    slotc_v[pl.ds(0, _L)] = plsc.load_gather(pos_v, [s0])
    slotc_v[pl.ds(_L, _L)] = plsc.load_gather(pos_v, [s1])
    pltpu.sync_copy(slotc_v, slotmap_out.at[pl.ds(rbase, 32)])


def _run_filter(edge_index, edge_type, sub, negones):
    mesh = plsc.VectorSubcoreMesh(
        core_axis_name="c", subcore_axis_name="s",
        num_cores=_NC, num_subcores=_NS)
    f = functools.partial(
        pl.kernel,
        out_type=[
            jax.ShapeDtypeStruct((_NW * _STRIDE,), jnp.int32),
            jax.ShapeDtypeStruct((_NW * _STRIDE,), jnp.int32),
            jax.ShapeDtypeStruct((_NW * _L,), jnp.int32),
            jax.ShapeDtypeStruct((_B,), jnp.int32),
        ],
        mesh=mesh,
        compiler_params=pltpu.CompilerParams(
            needs_layout_passes=False, use_tc_tiling_on_sc=False),
        scratch_types=[
            pltpu.VMEM((_NPOS,), jnp.int32),
            pltpu.VMEM((_B,), jnp.int32),
            pltpu.VMEM((_BLK,), jnp.int32),
            pltpu.VMEM((_BLK,), jnp.int32),
            pltpu.VMEM((_BLK,), jnp.int32),
            pltpu.VMEM((_STRIDE,), jnp.int32),
            pltpu.VMEM((_STRIDE,), jnp.int32),
            pltpu.VMEM((_L,), jnp.int32),
            pltpu.VMEM((32,), jnp.int32),
            pltpu.SemaphoreType.DMA,
            pltpu.SemaphoreType.DMA,
            pltpu.SemaphoreType.DMA,
        ],
    )(_filter_body)
    return f(edge_index, edge_type, sub, negones)


def _agg_body(srcf_hbm, psef_hbm, cnt_hbm, xpre_hbm, relx2_hbm, sub_hbm,
              zeros_hbm,
              agg_out, subx_out,
              srcall_v, psall_v, slot0_v, slot1_v, et0_v, et1_v,
              xrows0_v, xrows1_v, relrows0_v, relrows1_v, msg_v,
              cntv_v, sub32_v, subxc_v, agg_sh, gsem1, gsem2,
              xsem0, xsem1, rsem0, rsem1, ssem):
    cid = lax.axis_index("c")
    sid = lax.axis_index("s")
    wid = sid * _NC + cid
    iota = lax.iota(jnp.int32, _L)

    @pl.when(sid == 0)
    def _zero():
        pltpu.sync_copy(zeros_hbm, agg_sh)

    deg_vec = jnp.where(iota == 0, 1.0, 0.0).astype(jnp.float32)

    def _deg_init(e, c):
        msg_v[e, pl.ds(_FD, _L)] = deg_vec
        return c
    lax.fori_loop(0, _BATCH, _deg_init, jnp.int32(0))

    pltpu.sync_copy(cnt_hbm.at[pl.ds(wid * _L, _L)], cntv_v)

    rbase = wid * 32
    pltpu.sync_copy(sub_hbm.at[pl.ds(rbase, 32)], sub32_v)
    sx = pltpu.async_copy(xpre_hbm.at[sub32_v], subxc_v, ssem)

    plsc.subcore_barrier()

    cnt = cntv_v[pl.ds(0, _L)][0]
    hb = wid * _STRIDE
    nb = lax.div(cnt + jnp.int32(_BATCH - 1), jnp.int32(_BATCH))

    nblk = lax.div(cnt + jnp.int32(_BATCH + 2047), jnp.int32(2048))

    def _lst(b, c):
        c1 = pltpu.async_copy(srcf_hbm.at[pl.ds(hb + b * 2048, 2048)],
                              srcall_v.at[pl.ds(b * 2048, 2048)], gsem1)
        c2 = pltpu.async_copy(psef_hbm.at[pl.ds(hb + b * 2048, 2048)],
                              psall_v.at[pl.ds(b * 2048, 2048)], gsem2)
        c1.wait()
        c2.wait()
        return c
    lax.fori_loop(0, nblk, _lst, jnp.int32(0))

    slots = (slot0_v, slot1_v)
    ets = (et0_v, et1_v)
    xrows = (xrows0_v, xrows1_v)
    relrows = (relrows0_v, relrows1_v)
    xsems = (xsem0, xsem1)
    rsems = (rsem0, rsem1)

    def _unpack_issue(bi, buf):
        loff = bi * _BATCH

        def _unp(i, c2):
            v = psall_v[pl.ds(loff + i * _L, _L)]
            slots[buf][pl.ds(i * _L, _L)] = lax.shift_right_logical(v, 8)
            ets[buf][pl.ds(i * _L, _L)] = lax.bitwise_and(v, jnp.int32(255))
            return c2
        lax.fori_loop(0, _BATCH // _L, _unp, jnp.int32(0))
        pltpu.async_copy(xpre_hbm.at[srcall_v.at[pl.ds(loff, _BATCH)]],
                         xrows[buf], xsems[buf])
        pltpu.async_copy(relx2_hbm.at[ets[buf]], relrows[buf], rsems[buf])

    @pl.when(nb > 0)
    def _prologue():
        _unpack_issue(jnp.int32(0), 0)

    def _pair(pi, c):
        for b in range(2):
            i = pi * 2 + b
            nxt = 1 - b

            @pl.when(i < nb)
            def _step(i=i, b=b, nxt=nxt):
                @pl.when(i + 1 < nb)
                def _issue_next():
                    _unpack_issue(i + 1, nxt)

                pltpu.make_async_copy(
                    xpre_hbm.at[srcall_v.at[pl.ds(i * _BATCH, _BATCH)]],
                    xrows[b], xsems[b]).wait()
                pltpu.make_async_copy(
                    relx2_hbm.at[ets[b]], relrows[b], rsems[b]).wait()

                def _mul(e, c2):
                    for j in range(_FD // _L):
                        a = xrows[b][e, pl.ds(j * _L, _L)]
                        r = relrows[b][e, pl.ds(j * _L, _L)]
                        msg_v[e, pl.ds(j * _L, _L)] = a * r
                    return c2
                lax.fori_loop(0, _BATCH, _mul, jnp.int32(0), unroll=2)

                pltpu.sync_copy(msg_v, agg_sh.at[slots[b]], add=True)
        return c
    lax.fori_loop(0, lax.div(nb + jnp.int32(1), jnp.int32(2)), _pair,
                  jnp.int32(0))

    plsc.subcore_barrier()

    @pl.when(sid == 0)
    def _flush():
        pltpu.sync_copy(agg_sh, agg_out.at[cid])

    sx.wait()
    pltpu.sync_copy(subxc_v, subx_out.at[pl.ds(rbase, 32)])


def _run_agg(srcf, psef, cnts, xpre, relx2, sub, zeros):
    mesh = plsc.VectorSubcoreMesh(
        core_axis_name="c", subcore_axis_name="s",
        num_cores=_NC, num_subcores=_NS)
    f = functools.partial(
        pl.kernel,
        out_type=[
            jax.ShapeDtypeStruct((_NC, _AGG_ROWS, _WROW), jnp.float32),
            jax.ShapeDtypeStruct((_B, _FD), jnp.float32),
        ],
        mesh=mesh,
        compiler_params=pltpu.CompilerParams(
            needs_layout_passes=False, use_tc_tiling_on_sc=False),
        scratch_types=[
            pltpu.VMEM((_STRIDE,), jnp.int32),
            pltpu.VMEM((_STRIDE,), jnp.int32),
            pltpu.VMEM((_BATCH,), jnp.int32),
            pltpu.VMEM((_BATCH,), jnp.int32),
            pltpu.VMEM((_BATCH,), jnp.int32),
            pltpu.VMEM((_BATCH,), jnp.int32),
            pltpu.VMEM((_BATCH, _FD), jnp.float32),
            pltpu.VMEM((_BATCH, _FD), jnp.float32),
            pltpu.VMEM((_BATCH, _FD), jnp.float32),
            pltpu.VMEM((_BATCH, _FD), jnp.float32),
            pltpu.VMEM((_BATCH, _WROW), jnp.float32),
            pltpu.VMEM((_L,), jnp.int32),
            pltpu.VMEM((32,), jnp.int32),
            pltpu.VMEM((32, _FD), jnp.float32),
            pltpu.VMEM_SHARED((_AGG_ROWS, _WROW), jnp.float32),
            pltpu.SemaphoreType.DMA,
            pltpu.SemaphoreType.DMA,
            pltpu.SemaphoreType.DMA,
            pltpu.SemaphoreType.DMA,
            pltpu.SemaphoreType.DMA,
            pltpu.SemaphoreType.DMA,
            pltpu.SemaphoreType.DMA,
        ],
    )(_agg_body)
    return f(srcf, psef, cnts, xpre, relx2, sub, zeros)


def _club_body(agg2_ref, slot_ref, perm_ref, subx_ref,
               muW1, mub1, muW2, mub2, lvW1, lvb1, lvW2, lvb2, o_ref):
    agg = agg2_ref[0] + agg2_ref[1]
    oh = (slot_ref[...] == lax.broadcasted_iota(
        jnp.int32, (_B, _AGG_ROWS), 1)).astype(jnp.float32)
    sel = jnp.dot(oh, agg, preferred_element_type=jnp.float32,
                  precision=lax.Precision.HIGHEST)
    msg = sel[:, :_FD]
    deg = jnp.maximum(sel[:, _FD:_FD + 1], 1.0)
    xsub = jnp.tanh(subx_ref[...] + msg / deg)
    xs = xsub[:, :_D]
    ys = xsub[:, _D:_FD]
    hmu = jnp.maximum(jnp.dot(xs, muW1[...], preferred_element_type=jnp.float32,
                              precision=lax.Precision.DEFAULT)
                      + mub1[...], 0.0)
    mu = jnp.dot(hmu, muW2[...], preferred_element_type=jnp.float32,
                 precision=lax.Precision.DEFAULT) + mub2[...]
    hlv = jnp.maximum(jnp.dot(xs, lvW1[...], preferred_element_type=jnp.float32,
                              precision=lax.Precision.DEFAULT)
                      + lvb1[...], 0.0)
    logvar = jnp.tanh(jnp.dot(hlv, lvW2[...], preferred_element_type=jnp.float32,
                              precision=lax.Precision.DEFAULT)
                      + lvb2[...])
    inv = jnp.exp(-logvar)
    ohp = (perm_ref[...] == lax.broadcasted_iota(
        jnp.int32, (_B, _B), 1)).astype(jnp.float32)
    ysp = jnp.dot(ohp, ys, preferred_element_type=jnp.float32,
                  precision=lax.Precision.HIGHEST)
    pos_t = ((mu - ys) ** 2) * inv
    neg_t = ((mu - ysp) ** 2) * inv
    val = (jnp.sum(neg_t) - jnp.sum(pos_t)) / (2.0 * _B)
    o_ref[...] = val.reshape(1, 1)


def _run_club(agg2, slotmap, perm, subx,
              mu_W1, mu_b1, mu_W2, mu_b2, lv_W1, lv_b1, lv_W2, lv_b2):
    return pl.pallas_call(
        _club_body,
        out_shape=jax.ShapeDtypeStruct((1, 1), jnp.float32),
    )(agg2, slotmap.reshape(_B, 1), perm.reshape(_B, 1), subx,
      mu_W1, mu_b1.reshape(1, _H), mu_W2, mu_b2.reshape(1, _D),
      lv_W1, lv_b1.reshape(1, _H), lv_W2, lv_b2.reshape(1, _D))


def kernel(init_embed, pca_W, pca_b, rel, mu_W1, mu_b1, mu_W2, mu_b2,
           lv_W1, lv_b1, lv_W2, lv_b2, edge_index, edge_type, sub):
    negones = jnp.full((_NPOS,), -1, jnp.int32)
    zeros = jnp.zeros((_AGG_ROWS, _WROW), jnp.float32)
    relx2 = jnp.tile(rel, (1, _F))

    srcf, psef, cnts, slotmap = _run_filter(edge_index, edge_type, sub,
                                            negones)
    xpre = _compute_xpre(init_embed, pca_W, pca_b)
    agg2, subx = _run_agg(srcf, psef, cnts, xpre, relx2, sub, zeros)

    perm = jax.random.permutation(jax.random.key(1), _B).astype(jnp.int32)
    out = _run_club(agg2, slotmap, perm, subx,
                    mu_W1, mu_b1, mu_W2, mu_b2, lv_W1, lv_b1, lv_W2, lv_b2)
    return out.reshape(())
